# Initial kernel scaffold; baseline (speedup 1.0000x reference)
import functools

import jax
import jax.numpy as jnp
from jax import lax
from jax.experimental import pallas as pl
from jax.experimental.pallas import tpu as pltpu
from jax.experimental.pallas import tpu_sc as plsc

F32 = jnp.float32
I32 = jnp.int32
NEG = jnp.float32(-1e30)

B, S, H = 2, 2048, 1024
E, K = 16, 2
FF = 512
NG = 4
GS = E // NG
T = B * S
P = T * K
BM = 256
BMLOG = 8
NBLK = (P + E * (BM - 1) + BM - 1) // BM
MEXP = NBLK * BM
RB = 512
PB = 512
NPB = P // PB
NW = 32


def _routing_body(x_ref, gw_ref, logits_ref, tidx_ref, tw_ref):
    x = x_ref[...]
    gw = gw_ref[...]
    logits = lax.dot_general(x, gw, (((1,), (1,)), ((), ())),
                             preferred_element_type=F32)
    scores = 1.0 / (1.0 + jnp.exp(-logits))
    iota = lax.broadcasted_iota(I32, (RB, E), 1)
    grp = iota // GS

    m1 = jnp.full((RB, E), NEG)
    for g in range(NG):
        mg = jnp.max(jnp.where(grp == g, scores, NEG), axis=1, keepdims=True)
        m1 = jnp.where(grp == g, mg, m1)
    cand = jnp.where(scores == m1, iota, E)
    fst = jnp.zeros((RB, E), I32)
    for g in range(NG):
        fg = jnp.min(jnp.where(grp == g, cand, E), axis=1, keepdims=True)
        fst = jnp.where(grp == g, fg, fst)
    removed = iota == fst
    m2 = jnp.full((RB, E), NEG)
    for g in range(NG):
        mg = jnp.max(jnp.where((grp == g) & ~removed, scores, NEG),
                     axis=1, keepdims=True)
        m2 = jnp.where(grp == g, mg, m2)
    gsc = m1 + m2

    rep = iota % GS == 0
    gsc_rep = jnp.where(rep, gsc, NEG)
    g1v = jnp.max(gsc_rep, axis=1, keepdims=True)
    g1lane = jnp.min(jnp.where(gsc_rep == g1v, iota, E), axis=1, keepdims=True)
    gsc_rep2 = jnp.where(iota == g1lane, NEG, gsc_rep)
    g2v = jnp.max(gsc_rep2, axis=1, keepdims=True)
    g2lane = jnp.min(jnp.where(gsc_rep2 == g2v, iota, E), axis=1, keepdims=True)
    gmask = (grp == g1lane // GS) | (grp == g2lane // GS)

    masked = jnp.where(gmask, scores, NEG)
    v1 = jnp.max(masked, axis=1, keepdims=True)
    e1 = jnp.min(jnp.where(masked == v1, iota, E), axis=1, keepdims=True)
    masked2 = jnp.where(iota == e1, NEG, masked)
    v2 = jnp.max(masked2, axis=1, keepdims=True)
    e2 = jnp.min(jnp.where(masked2 == v2, iota, E), axis=1, keepdims=True)
    denom = v1 + v2 + 1e-20
    iota2 = lax.broadcasted_iota(I32, (RB, K), 1)
    logits_ref[...] = logits
    tidx_ref[...] = jnp.where(iota2 == 0, e1, e2)
    tw_ref[...] = jnp.where(iota2 == 0, v1 / denom, v2 / denom)


def _route(flat, gW):
    return pl.pallas_call(
        _routing_body,
        grid=(T // RB,),
        in_specs=[pl.BlockSpec((RB, H), lambda b: (b, 0)),
                  pl.BlockSpec((E, H), lambda b: (0, 0))],
        out_specs=[pl.BlockSpec((RB, E), lambda b: (b, 0)),
                   pl.BlockSpec((RB, K), lambda b: (b, 0)),
                   pl.BlockSpec((RB, K), lambda b: (b, 0))],
        out_shape=[jax.ShapeDtypeStruct((T, E), F32),
                   jax.ShapeDtypeStruct((T, K), I32),
                   jax.ShapeDtypeStruct((T, K), F32)],
    )(flat, gW)


def _dispatch_body(e_ref, pos_ref, be_ref, acc, bs, po, tot):
    s = pl.program_id(0)
    b = pl.program_id(1)
    e = e_ref[...]
    iota = lax.broadcasted_iota(I32, (PB, E), 1)
    oh = jnp.where(e == iota, 1.0, 0.0).astype(F32)

    @pl.when((s == 0) & (b == 0))
    def _():
        acc[...] = jnp.zeros_like(acc)

    @pl.when(s == 0)
    def _():
        bs[pl.ds(b, 1), :] = acc[0:1, :]
        acc[0:1, :] = acc[0:1, :] + jnp.sum(oh, axis=0, keepdims=True)

    @pl.when((s == 1) & (b == 0))
    def _():
        ci = acc[0:1, :].astype(I32)
        pc = ((ci + (BM - 1)) >> BMLOG) << BMLOG
        pcf = pc.astype(F32)
        r = lax.broadcasted_iota(I32, (E, E), 0)
        c = lax.broadcasted_iota(I32, (E, E), 1)
        m = jnp.where(r < c, 1.0, 0.0)
        po[0:1, :] = lax.dot_general(pcf, m, (((1,), (0,)), ((), ())),
                                     preferred_element_type=F32,
                                     precision=lax.Precision.HIGHEST)
        tot[0:1, :] = jnp.broadcast_to(jnp.sum(pcf, axis=1, keepdims=True),
                                       (1, E))

    @pl.when(s == 1)
    def _():
        r = lax.broadcasted_iota(I32, (PB, PB), 0)
        c = lax.broadcasted_iota(I32, (PB, PB), 1)
        tril = jnp.where(r > c, 1.0, 0.0)
        ranks = lax.dot_general(tril, oh, (((1,), (0,)), ((), ())),
                                preferred_element_type=F32,
                                precision=lax.Precision.HIGHEST)
        rank = jnp.sum(ranks * oh, axis=1, keepdims=True)
        base = jnp.sum(oh * (po[0:1, :] + bs[pl.ds(b, 1), :]),
                       axis=1, keepdims=True)
        pos_ref[...] = (rank + base).astype(I32)
        bi = lax.broadcasted_iota(I32, (NBLK, E), 0).astype(F32) * BM
        nle = jnp.sum(jnp.where(bi >= po[0:1, :], 1.0, 0.0),
                      axis=1, keepdims=True)
        bstart = lax.broadcasted_iota(I32, (NBLK, 1), 0).astype(F32) * BM
        be_ref[...] = jnp.where(bstart < tot[0:1, 0:1],
                                nle - 1.0, -1.0).astype(I32)


def _dispatch(e_col):
    return pl.pallas_call(
        _dispatch_body,
        grid=(2, NPB),
        in_specs=[pl.BlockSpec((PB, 1), lambda s, b: (b, 0))],
        out_specs=[pl.BlockSpec((PB, 1), lambda s, b: (b, 0)),
                   pl.BlockSpec((NBLK, 1), lambda s, b: (0, 0))],
        out_shape=[jax.ShapeDtypeStruct((P, 1), I32),
                   jax.ShapeDtypeStruct((NBLK, 1), I32)],
        scratch_shapes=[pltpu.VMEM((8, E), F32),
                        pltpu.VMEM((NPB, E), F32),
                        pltpu.VMEM((8, E), F32),
                        pltpu.VMEM((8, E), F32)],
    )(e_col)


_SC_MESH = plsc.VectorSubcoreMesh(core_axis_name="c", subcore_axis_name="s")
_C3 = 4
_C5 = 8


@functools.partial(
    pl.kernel, mesh=_SC_MESH,
    out_type=jax.ShapeDtypeStruct((MEXP, H), F32),
    scratch_types=[pltpu.VMEM((_C3, 32), I32),
                   pltpu.VMEM((_C3, 32), I32),
                   pltpu.VMEM((32, H), F32),
                   pltpu.SemaphoreType.DMA],
)
def _scatter(flat_hbm, pe_hbm, po_hbm, xs_hbm, idxe_v, idxo_v, rows_v, sem):
    wid = lax.axis_index("s") * 2 + lax.axis_index("c")
    pltpu.sync_copy(pe_hbm.at[wid], idxe_v)
    pltpu.sync_copy(po_hbm.at[wid], idxo_v)
    for c in range(_C3):
        base = wid * (T // NW) + c * 32
        pltpu.sync_copy(flat_hbm.at[pl.ds(base, 32)], rows_v)
        pltpu.async_copy(rows_v, xs_hbm.at[idxe_v.at[c]], sem).wait()
        pltpu.async_copy(rows_v, xs_hbm.at[idxo_v.at[c]], sem).wait()


def _mlp_block(x, gw, uw, dw):
    g = lax.dot_general(x, gw, (((1,), (1,)), ((), ())),
                        preferred_element_type=F32)
    u = lax.dot_general(x, uw, (((1,), (1,)), ((), ())),
                        preferred_element_type=F32)
    h = g / (1.0 + jnp.exp(-g)) * u
    return lax.dot_general(h, dw, (((1,), (1,)), ((), ())),
                           preferred_element_type=F32)


def _gmm_body(be_ref, x_ref, gw_ref, uw_ref, dw_ref, o_ref):
    b = pl.program_id(0)

    @pl.when(be_ref[b] >= 0)
    def _():
        o_ref[...] = _mlp_block(x_ref[...], gw_ref[0], uw_ref[0], dw_ref[0])


def _gmm(be, xs, egw, euw, edw):
    grid_spec = pltpu.PrefetchScalarGridSpec(
        num_scalar_prefetch=1,
        grid=(NBLK,),
        in_specs=[
            pl.BlockSpec((BM, H), lambda b, be_ref: (b, 0)),
            pl.BlockSpec((1, FF, H),
                         lambda b, be_ref: (jnp.maximum(be_ref[b], 0), 0, 0)),
            pl.BlockSpec((1, FF, H),
                         lambda b, be_ref: (jnp.maximum(be_ref[b], 0), 0, 0)),
            pl.BlockSpec((1, H, FF),
                         lambda b, be_ref: (jnp.maximum(be_ref[b], 0), 0, 0)),
        ],
        out_specs=pl.BlockSpec((BM, H), lambda b, be_ref: (b, 0)),
    )
    return pl.pallas_call(
        _gmm_body, grid_spec=grid_spec,
        out_shape=jax.ShapeDtypeStruct((MEXP, H), F32),
    )(be, xs, egw, euw, edw)


def _shared_body(x_ref, gw_ref, uw_ref, dw_ref, o_ref):
    o_ref[...] = _mlp_block(x_ref[...], gw_ref[...], uw_ref[...], dw_ref[...])


def _shared(flat, sgw, suw, sdw):
    return pl.pallas_call(
        _shared_body,
        grid=(T // BM,),
        in_specs=[pl.BlockSpec((BM, H), lambda b: (b, 0)),
                  pl.BlockSpec((FF, H), lambda b: (0, 0)),
                  pl.BlockSpec((FF, H), lambda b: (0, 0)),
                  pl.BlockSpec((H, FF), lambda b: (0, 0))],
        out_specs=pl.BlockSpec((BM, H), lambda b: (b, 0)),
        out_shape=jax.ShapeDtypeStruct((T, H), F32),
    )(flat, sgw, suw, sdw)


@functools.partial(
    pl.kernel, mesh=_SC_MESH,
    out_type=jax.ShapeDtypeStruct((T, H), F32),
    scratch_types=[pltpu.VMEM((_C5, 16), I32),
                   pltpu.VMEM((_C5, 16), I32),
                   pltpu.VMEM((_C5, 16), F32),
                   pltpu.VMEM((_C5, 16), F32),
                   pltpu.VMEM((16, H), F32),
                   pltpu.VMEM((16, H), F32),
                   pltpu.VMEM((16, H), F32),
                   pltpu.VMEM((16, H), F32),
                   pltpu.SemaphoreType.DMA],
)
def _combine(ys_hbm, ysh_hbm, pe_hbm, po_hbm, we_hbm, wo_hbm, y_hbm,
             idxe_v, idxo_v, we_v, wo_v, r0_v, r1_v, rsh_v, out_v, sem):
    wid = lax.axis_index("s") * 2 + lax.axis_index("c")
    pltpu.sync_copy(pe_hbm.at[wid], idxe_v)
    pltpu.sync_copy(po_hbm.at[wid], idxo_v)
    pltpu.sync_copy(we_hbm.at[wid], we_v)
    pltpu.sync_copy(wo_hbm.at[wid], wo_v)
    for c in range(_C5):
        base = wid * (T // NW) + c * 16
        pltpu.async_copy(ys_hbm.at[idxe_v.at[c]], r0_v, sem).wait()
        pltpu.async_copy(ys_hbm.at[idxo_v.at[c]], r1_v, sem).wait()
        pltpu.sync_copy(ysh_hbm.at[pl.ds(base, 16)], rsh_v)
        w0r = we_v[c, :]
        w1r = wo_v[c, :]
        for i in range(16):
            sel = jnp.zeros((16,), I32) + i
            b0 = jnp.take(w0r, sel)
            b1 = jnp.take(w1r, sel)

            def col(j, _):
                sl = pl.ds(j * 16, 16)
                out_v[i, sl] = b0 * r0_v[i, sl] + b1 * r1_v[i, sl] + rsh_v[i, sl]
                return 0

            lax.fori_loop(0, H // 16, col, 0, unroll=4)
        pltpu.sync_copy(out_v, y_hbm.at[pl.ds(base, 16)])


def kernel(hidden_states, gate_weight, expert_gate_w, expert_up_w,
           expert_down_w, shared_gate_w, shared_up_w, shared_down_w):
    flat = hidden_states.reshape(T, H)
    logits, tidx, tw = _route(flat, gate_weight)
    pos_col, be_col = _dispatch(tidx.reshape(P, 1))
    pos = pos_col.reshape(T, K)
    be = be_col.reshape(NBLK)
    pe3 = pos[:, 0].reshape(NW, _C3, 32)
    po3 = pos[:, 1].reshape(NW, _C3, 32)
    xs = _scatter(flat, pe3, po3)
    ys = _gmm(be, xs, expert_gate_w, expert_up_w, expert_down_w)
    ysh = _shared(flat, shared_gate_w, shared_up_w, shared_down_w)
    pe5 = pos[:, 0].reshape(NW, _C5, 16)
    po5 = pos[:, 1].reshape(NW, _C5, 16)
    w05 = tw[:, 0].reshape(NW, _C5, 16)
    w15 = tw[:, 1].reshape(NW, _C5, 16)
    y = _combine(ys, ysh, pe5, po5, w05, w15)
    return y.reshape(B, S, H), (logits.reshape(B, S, E), tidx.reshape(B, S, K))


# speedup vs baseline: 3.7548x; 3.7548x over previous
import functools

import jax
import jax.numpy as jnp
from jax import lax
from jax.experimental import pallas as pl
from jax.experimental.pallas import tpu as pltpu
from jax.experimental.pallas import tpu_sc as plsc

F32 = jnp.float32
I32 = jnp.int32
NEG = -1e30

B, S, H = 2, 2048, 1024
E, K = 16, 2
FF = 512
NG = 4
GS = E // NG
T = B * S
P = T * K
BM = 256
BMLOG = 8
NBLK = (P + E * (BM - 1) + BM - 1) // BM
MEXP = NBLK * BM
RB = 512
PB = 512
NPB = P // PB
NW = 32


def _routing_body(x_ref, gw_ref, logits_ref, tidx_ref, tw_ref):
    x = x_ref[...]
    gw = gw_ref[...]
    logits = lax.dot_general(x, gw, (((1,), (1,)), ((), ())),
                             preferred_element_type=F32)
    scores = 1.0 / (1.0 + jnp.exp(-logits))
    iota = lax.broadcasted_iota(I32, (RB, E), 1)
    grp = iota // GS

    m1 = jnp.full((RB, E), NEG)
    for g in range(NG):
        mg = jnp.max(jnp.where(grp == g, scores, NEG), axis=1, keepdims=True)
        m1 = jnp.where(grp == g, mg, m1)
    cand = jnp.where(scores == m1, iota, E)
    fst = jnp.zeros((RB, E), I32)
    for g in range(NG):
        fg = jnp.min(jnp.where(grp == g, cand, E), axis=1, keepdims=True)
        fst = jnp.where(grp == g, fg, fst)
    removed = iota == fst
    m2 = jnp.full((RB, E), NEG)
    for g in range(NG):
        mg = jnp.max(jnp.where((grp == g) & ~removed, scores, NEG),
                     axis=1, keepdims=True)
        m2 = jnp.where(grp == g, mg, m2)
    gsc = m1 + m2

    rep = iota % GS == 0
    gsc_rep = jnp.where(rep, gsc, NEG)
    g1v = jnp.max(gsc_rep, axis=1, keepdims=True)
    g1lane = jnp.min(jnp.where(gsc_rep == g1v, iota, E), axis=1, keepdims=True)
    gsc_rep2 = jnp.where(iota == g1lane, NEG, gsc_rep)
    g2v = jnp.max(gsc_rep2, axis=1, keepdims=True)
    g2lane = jnp.min(jnp.where(gsc_rep2 == g2v, iota, E), axis=1, keepdims=True)
    gmask = (grp == g1lane // GS) | (grp == g2lane // GS)

    masked = jnp.where(gmask, scores, NEG)
    v1 = jnp.max(masked, axis=1, keepdims=True)
    e1 = jnp.min(jnp.where(masked == v1, iota, E), axis=1, keepdims=True)
    masked2 = jnp.where(iota == e1, NEG, masked)
    v2 = jnp.max(masked2, axis=1, keepdims=True)
    e2 = jnp.min(jnp.where(masked2 == v2, iota, E), axis=1, keepdims=True)
    denom = v1 + v2 + 1e-20
    iota2 = lax.broadcasted_iota(I32, (RB, K), 1)
    logits_ref[...] = logits
    tidx_ref[...] = jnp.where(iota2 == 0, e1, e2)
    tw_ref[...] = jnp.where(iota2 == 0, v1 / denom, v2 / denom)


def _route(flat, gW):
    return pl.pallas_call(
        _routing_body,
        grid=(T // RB,),
        in_specs=[pl.BlockSpec((RB, H), lambda b: (b, 0)),
                  pl.BlockSpec((E, H), lambda b: (0, 0))],
        out_specs=[pl.BlockSpec((RB, E), lambda b: (b, 0)),
                   pl.BlockSpec((RB, K), lambda b: (b, 0)),
                   pl.BlockSpec((RB, K), lambda b: (b, 0))],
        out_shape=[jax.ShapeDtypeStruct((T, E), F32),
                   jax.ShapeDtypeStruct((T, K), I32),
                   jax.ShapeDtypeStruct((T, K), F32)],
    )(flat, gW)


def _dispatch_body(e_ref, pos_ref, be_ref, acc, bs, po, tot):
    s = pl.program_id(0)
    b = pl.program_id(1)
    e = e_ref[...]
    iota = lax.broadcasted_iota(I32, (PB, E), 1)
    oh = jnp.where(e == iota, 1.0, 0.0).astype(F32)

    @pl.when((s == 0) & (b == 0))
    def _():
        acc[...] = jnp.zeros_like(acc)

    @pl.when(s == 0)
    def _():
        bs[pl.ds(b, 1), :] = acc[0:1, :]
        acc[0:1, :] = acc[0:1, :] + jnp.sum(oh, axis=0, keepdims=True)

    @pl.when((s == 1) & (b == 0))
    def _():
        ci = acc[0:1, :].astype(I32)
        pc = ((ci + (BM - 1)) >> BMLOG) << BMLOG
        pcf = pc.astype(F32)
        r = lax.broadcasted_iota(I32, (E, E), 0)
        c = lax.broadcasted_iota(I32, (E, E), 1)
        m = jnp.where(r < c, 1.0, 0.0)
        po[0:1, :] = lax.dot_general(pcf, m, (((1,), (0,)), ((), ())),
                                     preferred_element_type=F32,
                                     precision=lax.Precision.HIGHEST)
        tot[0:1, :] = jnp.broadcast_to(jnp.sum(pcf, axis=1, keepdims=True),
                                       (1, E))

    @pl.when(s == 1)
    def _():
        r = lax.broadcasted_iota(I32, (PB, PB), 0)
        c = lax.broadcasted_iota(I32, (PB, PB), 1)
        tril = jnp.where(r > c, 1.0, 0.0)
        ranks = lax.dot_general(tril, oh, (((1,), (0,)), ((), ())),
                                preferred_element_type=F32,
                                precision=lax.Precision.HIGHEST)
        rank = jnp.sum(ranks * oh, axis=1, keepdims=True)
        base = jnp.sum(oh * (po[0:1, :] + bs[pl.ds(b, 1), :]),
                       axis=1, keepdims=True)
        pos_ref[...] = (rank + base).astype(I32)
        bi = lax.broadcasted_iota(I32, (NBLK, E), 0).astype(F32) * BM
        nle = jnp.sum(jnp.where(bi >= po[0:1, :], 1.0, 0.0),
                      axis=1, keepdims=True)
        bstart = lax.broadcasted_iota(I32, (NBLK, 1), 0).astype(F32) * BM
        be_ref[...] = jnp.where(bstart < tot[0:1, 0:1],
                                nle - 1.0, -1.0).astype(I32)


def _dispatch(e_col):
    return pl.pallas_call(
        _dispatch_body,
        grid=(2, NPB),
        in_specs=[pl.BlockSpec((PB, 1), lambda s, b: (b, 0))],
        out_specs=[pl.BlockSpec((PB, 1), lambda s, b: (b, 0)),
                   pl.BlockSpec((NBLK, 1), lambda s, b: (0, 0))],
        out_shape=[jax.ShapeDtypeStruct((P, 1), I32),
                   jax.ShapeDtypeStruct((NBLK, 1), I32)],
        scratch_shapes=[pltpu.VMEM((8, E), F32),
                        pltpu.VMEM((NPB, E), F32),
                        pltpu.VMEM((8, E), F32),
                        pltpu.VMEM((8, E), F32)],
    )(e_col)


_C3 = 4
_C5 = 8


@functools.cache
def _sc_mesh():
    return plsc.VectorSubcoreMesh(core_axis_name="c", subcore_axis_name="s",
                                  num_cores=2, num_subcores=16)


@functools.cache
def _build_scatter():
    @functools.partial(
        pl.kernel, mesh=_sc_mesh(),
        out_type=jax.ShapeDtypeStruct((MEXP, H), F32),
        scratch_types=[pltpu.VMEM((_C3, 32), I32),
                       pltpu.VMEM((_C3, 32), I32),
                       pltpu.VMEM((32, H), F32),
                       pltpu.SemaphoreType.DMA],
    )
    def scatter_kernel(flat_hbm, pe_hbm, po_hbm, xs_hbm,
                       idxe_v, idxo_v, rows_v, sem):
        wid = lax.axis_index("s") * 2 + lax.axis_index("c")
        pltpu.sync_copy(pe_hbm.at[wid], idxe_v)
        pltpu.sync_copy(po_hbm.at[wid], idxo_v)
        for c in range(_C3):
            base = wid * (T // NW) + c * 32
            pltpu.sync_copy(flat_hbm.at[pl.ds(base, 32)], rows_v)
            pltpu.async_copy(rows_v, xs_hbm.at[idxe_v.at[c]], sem).wait()
            pltpu.async_copy(rows_v, xs_hbm.at[idxo_v.at[c]], sem).wait()

    return scatter_kernel


def _scatter(flat, pe3, po3):
    return _build_scatter()(flat, pe3, po3)


def _mlp_block(x, gw, uw, dw):
    g = lax.dot_general(x, gw, (((1,), (1,)), ((), ())),
                        preferred_element_type=F32)
    u = lax.dot_general(x, uw, (((1,), (1,)), ((), ())),
                        preferred_element_type=F32)
    h = g / (1.0 + jnp.exp(-g)) * u
    return lax.dot_general(h, dw, (((1,), (1,)), ((), ())),
                           preferred_element_type=F32)


def _gmm_body(be_ref, x_ref, gw_ref, uw_ref, dw_ref, o_ref):
    b = pl.program_id(0)

    @pl.when(be_ref[b] >= 0)
    def _():
        o_ref[...] = _mlp_block(x_ref[...], gw_ref[0], uw_ref[0], dw_ref[0])


def _gmm(be, xs, egw, euw, edw):
    grid_spec = pltpu.PrefetchScalarGridSpec(
        num_scalar_prefetch=1,
        grid=(NBLK,),
        in_specs=[
            pl.BlockSpec((BM, H), lambda b, be_ref: (b, 0)),
            pl.BlockSpec((1, FF, H),
                         lambda b, be_ref: (jnp.maximum(be_ref[b], 0), 0, 0)),
            pl.BlockSpec((1, FF, H),
                         lambda b, be_ref: (jnp.maximum(be_ref[b], 0), 0, 0)),
            pl.BlockSpec((1, H, FF),
                         lambda b, be_ref: (jnp.maximum(be_ref[b], 0), 0, 0)),
        ],
        out_specs=pl.BlockSpec((BM, H), lambda b, be_ref: (b, 0)),
    )
    return pl.pallas_call(
        _gmm_body, grid_spec=grid_spec,
        out_shape=jax.ShapeDtypeStruct((MEXP, H), F32),
    )(be, xs, egw, euw, edw)


def _shared_body(x_ref, gw_ref, uw_ref, dw_ref, o_ref):
    o_ref[...] = _mlp_block(x_ref[...], gw_ref[...], uw_ref[...], dw_ref[...])


def _shared(flat, sgw, suw, sdw):
    return pl.pallas_call(
        _shared_body,
        grid=(T // BM,),
        in_specs=[pl.BlockSpec((BM, H), lambda b: (b, 0)),
                  pl.BlockSpec((FF, H), lambda b: (0, 0)),
                  pl.BlockSpec((FF, H), lambda b: (0, 0)),
                  pl.BlockSpec((H, FF), lambda b: (0, 0))],
        out_specs=pl.BlockSpec((BM, H), lambda b: (b, 0)),
        out_shape=jax.ShapeDtypeStruct((T, H), F32),
    )(flat, sgw, suw, sdw)


@functools.cache
def _build_combine():
    @functools.partial(
        pl.kernel, mesh=_sc_mesh(),
        out_type=jax.ShapeDtypeStruct((T, H), F32),
        scratch_types=[pltpu.VMEM((_C5, 16), I32),
                       pltpu.VMEM((_C5, 16), I32),
                       pltpu.VMEM((_C5, 16), F32),
                       pltpu.VMEM((_C5, 16), F32),
                       pltpu.VMEM((16, H), F32),
                       pltpu.VMEM((16, H), F32),
                       pltpu.VMEM((16, H), F32),
                       pltpu.VMEM((16, H), F32),
                       pltpu.SemaphoreType.DMA],
    )
    def combine_kernel(ys_hbm, ysh_hbm, pe_hbm, po_hbm, we_hbm, wo_hbm, y_hbm,
                       idxe_v, idxo_v, we_v, wo_v, r0_v, r1_v, rsh_v, out_v,
                       sem):
        wid = lax.axis_index("s") * 2 + lax.axis_index("c")
        pltpu.sync_copy(pe_hbm.at[wid], idxe_v)
        pltpu.sync_copy(po_hbm.at[wid], idxo_v)
        pltpu.sync_copy(we_hbm.at[wid], we_v)
        pltpu.sync_copy(wo_hbm.at[wid], wo_v)
        for c in range(_C5):
            base = wid * (T // NW) + c * 16
            pltpu.async_copy(ys_hbm.at[idxe_v.at[c]], r0_v, sem).wait()
            pltpu.async_copy(ys_hbm.at[idxo_v.at[c]], r1_v, sem).wait()
            pltpu.sync_copy(ysh_hbm.at[pl.ds(base, 16)], rsh_v)
            w0r = we_v[c, :]
            w1r = wo_v[c, :]
            for i in range(16):
                sel = jnp.zeros((16,), I32) + i
                b0 = jnp.take(w0r, sel)
                b1 = jnp.take(w1r, sel)

                def col(j, _):
                    sl = pl.ds(j * 16, 16)
                    out_v[i, sl] = (b0 * r0_v[i, sl] + b1 * r1_v[i, sl]
                                    + rsh_v[i, sl])
                    return 0

                lax.fori_loop(0, H // 16, col, 0, unroll=4)
            pltpu.sync_copy(out_v, y_hbm.at[pl.ds(base, 16)])

    return combine_kernel


def _combine(ys, ysh, pe5, po5, w05, w15):
    return _build_combine()(ys, ysh, pe5, po5, w05, w15)


def kernel(hidden_states, gate_weight, expert_gate_w, expert_up_w,
           expert_down_w, shared_gate_w, shared_up_w, shared_down_w):
    flat = hidden_states.reshape(T, H)
    logits, tidx, tw = _route(flat, gate_weight)
    pos_col, be_col = _dispatch(tidx.reshape(P, 1))
    pos = pos_col.reshape(T, K)
    be = be_col.reshape(NBLK)
    pe3 = pos[:, 0].reshape(NW, _C3, 32)
    po3 = pos[:, 1].reshape(NW, _C3, 32)
    xs = _scatter(flat, pe3, po3)
    ys = _gmm(be, xs, expert_gate_w, expert_up_w, expert_down_w)
    ysh = _shared(flat, shared_gate_w, shared_up_w, shared_down_w)
    pe5 = pos[:, 0].reshape(NW, _C5, 16)
    po5 = pos[:, 1].reshape(NW, _C5, 16)
    w05 = tw[:, 0].reshape(NW, _C5, 16)
    w15 = tw[:, 1].reshape(NW, _C5, 16)
    y = _combine(ys, ysh, pe5, po5, w05, w15)
    return y.reshape(B, S, H), (logits.reshape(B, S, E), tidx.reshape(B, S, K))


# speedup vs baseline: 3.9053x; 1.0401x over previous
import functools

import jax
import jax.numpy as jnp
from jax import lax
from jax.experimental import pallas as pl
from jax.experimental.pallas import tpu as pltpu
from jax.experimental.pallas import tpu_sc as plsc

F32 = jnp.float32
I32 = jnp.int32
NEG = -1e30

B, S, H = 2, 2048, 1024
E, K = 16, 2
FF = 512
NG = 4
GS = E // NG
T = B * S
P = T * K
BM = 256
BMLOG = 8
NBLK = (P + E * (BM - 1) + BM - 1) // BM
MEXP = NBLK * BM
RB = 512
PB = 512
NPB = P // PB
NW = 32


def _routing_body(x_ref, gw_ref, logits_ref, tidx_ref, tw_ref):
    x = x_ref[...]
    gw = gw_ref[...]
    logits = lax.dot_general(x, gw, (((1,), (1,)), ((), ())),
                             preferred_element_type=F32)
    scores = 1.0 / (1.0 + jnp.exp(-logits))
    iota = lax.broadcasted_iota(I32, (RB, E), 1)
    grp = iota // GS

    m1 = jnp.full((RB, E), NEG)
    for g in range(NG):
        mg = jnp.max(jnp.where(grp == g, scores, NEG), axis=1, keepdims=True)
        m1 = jnp.where(grp == g, mg, m1)
    cand = jnp.where(scores == m1, iota, E)
    fst = jnp.zeros((RB, E), I32)
    for g in range(NG):
        fg = jnp.min(jnp.where(grp == g, cand, E), axis=1, keepdims=True)
        fst = jnp.where(grp == g, fg, fst)
    removed = iota == fst
    m2 = jnp.full((RB, E), NEG)
    for g in range(NG):
        mg = jnp.max(jnp.where((grp == g) & ~removed, scores, NEG),
                     axis=1, keepdims=True)
        m2 = jnp.where(grp == g, mg, m2)
    gsc = m1 + m2

    rep = iota % GS == 0
    gsc_rep = jnp.where(rep, gsc, NEG)
    g1v = jnp.max(gsc_rep, axis=1, keepdims=True)
    g1lane = jnp.min(jnp.where(gsc_rep == g1v, iota, E), axis=1, keepdims=True)
    gsc_rep2 = jnp.where(iota == g1lane, NEG, gsc_rep)
    g2v = jnp.max(gsc_rep2, axis=1, keepdims=True)
    g2lane = jnp.min(jnp.where(gsc_rep2 == g2v, iota, E), axis=1, keepdims=True)
    gmask = (grp == g1lane // GS) | (grp == g2lane // GS)

    masked = jnp.where(gmask, scores, NEG)
    v1 = jnp.max(masked, axis=1, keepdims=True)
    e1 = jnp.min(jnp.where(masked == v1, iota, E), axis=1, keepdims=True)
    masked2 = jnp.where(iota == e1, NEG, masked)
    v2 = jnp.max(masked2, axis=1, keepdims=True)
    e2 = jnp.min(jnp.where(masked2 == v2, iota, E), axis=1, keepdims=True)
    denom = v1 + v2 + 1e-20
    iota2 = lax.broadcasted_iota(I32, (RB, K), 1)
    logits_ref[...] = logits
    tidx_ref[...] = jnp.where(iota2 == 0, e1, e2)
    tw_ref[...] = jnp.where(iota2 == 0, v1 / denom, v2 / denom)


def _route(flat, gW):
    return pl.pallas_call(
        _routing_body,
        grid=(T // RB,),
        in_specs=[pl.BlockSpec((RB, H), lambda b: (b, 0)),
                  pl.BlockSpec((E, H), lambda b: (0, 0))],
        out_specs=[pl.BlockSpec((RB, E), lambda b: (b, 0)),
                   pl.BlockSpec((RB, K), lambda b: (b, 0)),
                   pl.BlockSpec((RB, K), lambda b: (b, 0))],
        out_shape=[jax.ShapeDtypeStruct((T, E), F32),
                   jax.ShapeDtypeStruct((T, K), I32),
                   jax.ShapeDtypeStruct((T, K), F32)],
    )(flat, gW)


def _dispatch_body(e_ref, pos_ref, be_ref, acc, bs, po, tot, tril_s):
    s = pl.program_id(0)
    b = pl.program_id(1)
    e = e_ref[...]
    iota = lax.broadcasted_iota(I32, (PB, E), 1)
    oh = jnp.where(e == iota, 1.0, 0.0).astype(F32)

    @pl.when((s == 0) & (b == 0))
    def _():
        acc[...] = jnp.zeros_like(acc)
        r = lax.broadcasted_iota(I32, (PB, PB), 0)
        c = lax.broadcasted_iota(I32, (PB, PB), 1)
        tril_s[...] = jnp.where(r > c, 1.0, 0.0)

    @pl.when(s == 0)
    def _():
        bs[pl.ds(b, 1), :] = acc[0:1, :]
        acc[0:1, :] = acc[0:1, :] + jnp.sum(oh, axis=0, keepdims=True)

    @pl.when((s == 1) & (b == 0))
    def _():
        ci = acc[0:1, :].astype(I32)
        pc = ((ci + (BM - 1)) >> BMLOG) << BMLOG
        pcf = pc.astype(F32)
        r = lax.broadcasted_iota(I32, (E, E), 0)
        c = lax.broadcasted_iota(I32, (E, E), 1)
        m = jnp.where(r < c, 1.0, 0.0)
        po[0:1, :] = lax.dot_general(pcf, m, (((1,), (0,)), ((), ())),
                                     preferred_element_type=F32,
                                     precision=lax.Precision.HIGHEST)
        tot[0:1, :] = jnp.broadcast_to(jnp.sum(pcf, axis=1, keepdims=True),
                                       (1, E))

    @pl.when(s == 1)
    def _():
        ranks = lax.dot_general(tril_s[...], oh, (((1,), (0,)), ((), ())),
                                preferred_element_type=F32,
                                precision=lax.Precision.HIGHEST)
        rank = jnp.sum(ranks * oh, axis=1, keepdims=True)
        base = jnp.sum(oh * (po[0:1, :] + bs[pl.ds(b, 1), :]),
                       axis=1, keepdims=True)
        pos_ref[...] = (rank + base).astype(I32)
        bi = lax.broadcasted_iota(I32, (NBLK, E), 0).astype(F32) * BM
        nle = jnp.sum(jnp.where(bi >= po[0:1, :], 1.0, 0.0),
                      axis=1, keepdims=True)
        bstart = lax.broadcasted_iota(I32, (NBLK, 1), 0).astype(F32) * BM
        be_ref[...] = jnp.where(bstart < tot[0:1, 0:1],
                                nle - 1.0, -1.0).astype(I32)


def _dispatch(e_col):
    return pl.pallas_call(
        _dispatch_body,
        grid=(2, NPB),
        in_specs=[pl.BlockSpec((PB, 1), lambda s, b: (b, 0))],
        out_specs=[pl.BlockSpec((PB, 1), lambda s, b: (b, 0)),
                   pl.BlockSpec((NBLK, 1), lambda s, b: (0, 0))],
        out_shape=[jax.ShapeDtypeStruct((P, 1), I32),
                   jax.ShapeDtypeStruct((NBLK, 1), I32)],
        scratch_shapes=[pltpu.VMEM((8, E), F32),
                        pltpu.VMEM((NPB, E), F32),
                        pltpu.VMEM((8, E), F32),
                        pltpu.VMEM((8, E), F32),
                        pltpu.VMEM((PB, PB), F32)],
    )(e_col)


_C3 = 4


@functools.cache
def _sc_mesh():
    return plsc.VectorSubcoreMesh(core_axis_name="c", subcore_axis_name="s",
                                  num_cores=2, num_subcores=16)


@functools.cache
def _build_scatter():
    @functools.partial(
        pl.kernel, mesh=_sc_mesh(),
        out_type=jax.ShapeDtypeStruct((MEXP, H), F32),
        scratch_types=[pltpu.VMEM((_C3, 32), I32),
                       pltpu.VMEM((_C3, 32), I32),
                       pltpu.VMEM((32, H), F32),
                       pltpu.SemaphoreType.DMA],
    )
    def scatter_kernel(flat_hbm, pe_hbm, po_hbm, xs_hbm,
                       idxe_v, idxo_v, rows_v, sem):
        wid = lax.axis_index("s") * 2 + lax.axis_index("c")
        pltpu.sync_copy(pe_hbm.at[wid], idxe_v)
        pltpu.sync_copy(po_hbm.at[wid], idxo_v)
        for c in range(_C3):
            base = wid * (T // NW) + c * 32
            pltpu.sync_copy(flat_hbm.at[pl.ds(base, 32)], rows_v)
            pltpu.async_copy(rows_v, xs_hbm.at[idxe_v.at[c]], sem).wait()
            pltpu.async_copy(rows_v, xs_hbm.at[idxo_v.at[c]], sem).wait()

    return scatter_kernel


def _scatter(flat, pe3, po3):
    return _build_scatter()(flat, pe3, po3)


def _mlp_block(x, gw, uw, dw):
    xb = x.astype(jnp.bfloat16)
    g = lax.dot_general(xb, gw.astype(jnp.bfloat16), (((1,), (1,)), ((), ())),
                        preferred_element_type=F32)
    u = lax.dot_general(xb, uw.astype(jnp.bfloat16), (((1,), (1,)), ((), ())),
                        preferred_element_type=F32)
    h = (g / (1.0 + jnp.exp(-g)) * u).astype(jnp.bfloat16)
    return lax.dot_general(h, dw.astype(jnp.bfloat16), (((1,), (1,)), ((), ())),
                           preferred_element_type=F32)


def _gmm_body(be_ref, x_ref, gw_ref, uw_ref, dw_ref, o_ref):
    b = pl.program_id(0)

    @pl.when(be_ref[b] >= 0)
    def _():
        o_ref[...] = _mlp_block(x_ref[...], gw_ref[0], uw_ref[0], dw_ref[0])


def _gmm(be, xs, egw, euw, edw):
    grid_spec = pltpu.PrefetchScalarGridSpec(
        num_scalar_prefetch=1,
        grid=(NBLK,),
        in_specs=[
            pl.BlockSpec((BM, H), lambda b, be_ref: (b, 0)),
            pl.BlockSpec((1, FF, H),
                         lambda b, be_ref: (jnp.maximum(be_ref[b], 0), 0, 0)),
            pl.BlockSpec((1, FF, H),
                         lambda b, be_ref: (jnp.maximum(be_ref[b], 0), 0, 0)),
            pl.BlockSpec((1, H, FF),
                         lambda b, be_ref: (jnp.maximum(be_ref[b], 0), 0, 0)),
        ],
        out_specs=pl.BlockSpec((BM, H), lambda b, be_ref: (b, 0)),
    )
    return pl.pallas_call(
        _gmm_body, grid_spec=grid_spec,
        out_shape=jax.ShapeDtypeStruct((MEXP, H), F32),
    )(be, xs, egw, euw, edw)


def _shared_body(x_ref, gw_ref, uw_ref, dw_ref, o_ref):
    o_ref[...] = _mlp_block(x_ref[...], gw_ref[...], uw_ref[...], dw_ref[...])


def _shared(flat, sgw, suw, sdw):
    return pl.pallas_call(
        _shared_body,
        grid=(T // BM,),
        in_specs=[pl.BlockSpec((BM, H), lambda b: (b, 0)),
                  pl.BlockSpec((FF, H), lambda b: (0, 0)),
                  pl.BlockSpec((FF, H), lambda b: (0, 0)),
                  pl.BlockSpec((H, FF), lambda b: (0, 0))],
        out_specs=pl.BlockSpec((BM, H), lambda b: (b, 0)),
        out_shape=jax.ShapeDtypeStruct((T, H), F32),
    )(flat, sgw, suw, sdw)


_CT = 8
_C5 = (T // NW) // _CT


@functools.cache
def _build_combine():
    @functools.partial(
        pl.kernel, mesh=_sc_mesh(),
        out_type=jax.ShapeDtypeStruct((T, H), F32),
        scratch_types=[pltpu.VMEM((_C5, _CT), I32),
                       pltpu.VMEM((_C5, _CT), I32),
                       pltpu.VMEM((_C5 * _CT // 16, 16), F32),
                       pltpu.VMEM((_C5 * _CT // 16, 16), F32),
                       pltpu.VMEM((2, _CT, H), F32),
                       pltpu.VMEM((2, _CT, H), F32),
                       pltpu.VMEM((2, _CT, H), F32),
                       pltpu.VMEM((_CT, H), F32),
                       pltpu.SemaphoreType.DMA,
                       pltpu.SemaphoreType.DMA],
    )
    def combine_kernel(ys_hbm, ysh_hbm, pe_hbm, po_hbm, we_hbm, wo_hbm, y_hbm,
                       idxe_v, idxo_v, we_v, wo_v, r0_v, r1_v, rsh_v, out_v,
                       sem0, sem1):
        wid = lax.axis_index("s") * 2 + lax.axis_index("c")
        pltpu.sync_copy(pe_hbm.at[wid], idxe_v)
        pltpu.sync_copy(po_hbm.at[wid], idxo_v)
        pltpu.sync_copy(we_hbm.at[wid], we_v)
        pltpu.sync_copy(wo_hbm.at[wid], wo_v)
        sems = (sem0, sem1)

        def issue(c, slot):
            base = wid * (T // NW) + c * _CT
            return (
                pltpu.async_copy(ys_hbm.at[idxe_v.at[c]], r0_v.at[slot],
                                 sems[slot]),
                pltpu.async_copy(ys_hbm.at[idxo_v.at[c]], r1_v.at[slot],
                                 sems[slot]),
                pltpu.async_copy(ysh_hbm.at[pl.ds(base, _CT)], rsh_v.at[slot],
                                 sems[slot]),
            )

        handles = [issue(0, 0), None]
        for c in range(_C5):
            slot = c % 2
            if c + 1 < _C5:
                handles[1 - slot] = issue(c + 1, 1 - slot)
            for h in handles[slot]:
                h.wait()
            base = wid * (T // NW) + c * _CT
            w0r = we_v[c // 2, :]
            w1r = wo_v[c // 2, :]
            off = (c % 2) * _CT

            def tok(i, _):
                sel = jnp.zeros((16,), I32) + (off + i)
                b0 = jnp.take(w0r, sel)
                b1 = jnp.take(w1r, sel)

                def col(j, _):
                    sl = pl.ds(j * 16, 16)
                    out_v[i, sl] = (b0 * r0_v[slot, i, sl]
                                    + b1 * r1_v[slot, i, sl]
                                    + rsh_v[slot, i, sl])
                    return 0

                lax.fori_loop(0, H // 16, col, 0, unroll=8)
                return 0

            lax.fori_loop(0, _CT, tok, 0)
            pltpu.sync_copy(out_v, y_hbm.at[pl.ds(base, _CT)])

    return combine_kernel


def _combine(ys, ysh, pe5, po5, w05, w15):
    return _build_combine()(ys, ysh, pe5, po5, w05, w15)


def kernel(hidden_states, gate_weight, expert_gate_w, expert_up_w,
           expert_down_w, shared_gate_w, shared_up_w, shared_down_w):
    flat = hidden_states.reshape(T, H)
    logits, tidx, tw = _route(flat, gate_weight)
    pos_col, be_col = _dispatch(tidx.reshape(P, 1))
    pos = pos_col.reshape(T, K)
    be = be_col.reshape(NBLK)
    pe3 = pos[:, 0].reshape(NW, _C3, 32)
    po3 = pos[:, 1].reshape(NW, _C3, 32)
    xs = _scatter(flat, pe3, po3)
    ys = _gmm(be, xs, expert_gate_w, expert_up_w, expert_down_w)
    ysh = _shared(flat, shared_gate_w, shared_up_w, shared_down_w)
    pe5 = pos[:, 0].reshape(NW, _C5, _CT)
    po5 = pos[:, 1].reshape(NW, _C5, _CT)
    w05 = tw[:, 0].reshape(NW, _C5 * _CT // 16, 16)
    w15 = tw[:, 1].reshape(NW, _C5 * _CT // 16, 16)
    y = _combine(ys, ysh, pe5, po5, w05, w15)
    return y.reshape(B, S, H), (logits.reshape(B, S, E), tidx.reshape(B, S, K))


# speedup vs baseline: 4.4466x; 1.1386x over previous
import functools

import jax
import jax.numpy as jnp
from jax import lax
from jax.experimental import pallas as pl
from jax.experimental.pallas import tpu as pltpu
from jax.experimental.pallas import tpu_sc as plsc

F32 = jnp.float32
I32 = jnp.int32
NEG = -1e30

B, S, H = 2, 2048, 1024
E, K = 16, 2
FF = 512
NG = 4
GS = E // NG
T = B * S
P = T * K
BM = 256
BMLOG = 8
NBLK = (P + E * (BM - 1) + BM - 1) // BM
MEXP = NBLK * BM
RB = 512
PB = 512
NPB = P // PB
NW = 32
HW = H // 2


def _pack_row(xlo, xhi):
    ulo = lax.bitcast_convert_type(xlo, jnp.uint32)
    uhi = lax.bitcast_convert_type(xhi, jnp.uint32)
    rlo = ulo + jnp.uint32(0x7FFF) + ((ulo >> 16) & jnp.uint32(1))
    rhi = uhi + jnp.uint32(0x7FFF) + ((uhi >> 16) & jnp.uint32(1))
    w = (rlo >> 16) | (rhi & jnp.uint32(0xFFFF0000))
    return lax.bitcast_convert_type(w, I32)


def _unpack_row(w):
    lo = lax.bitcast_convert_type(w << 16, F32)
    hi = lax.bitcast_convert_type(w & jnp.int32(-65536), F32)
    return lo, hi


def _routing_body(x_ref, gw_ref, logits_ref, tidx_ref, tw_ref, xbf_ref):
    x = x_ref[...]
    gw = gw_ref[...]
    logits = lax.dot_general(x, gw, (((1,), (1,)), ((), ())),
                             preferred_element_type=F32)
    scores = 1.0 / (1.0 + jnp.exp(-logits))
    iota = lax.broadcasted_iota(I32, (RB, E), 1)
    grp = iota // GS

    m1 = jnp.full((RB, E), NEG)
    for g in range(NG):
        mg = jnp.max(jnp.where(grp == g, scores, NEG), axis=1, keepdims=True)
        m1 = jnp.where(grp == g, mg, m1)
    cand = jnp.where(scores == m1, iota, E)
    fst = jnp.zeros((RB, E), I32)
    for g in range(NG):
        fg = jnp.min(jnp.where(grp == g, cand, E), axis=1, keepdims=True)
        fst = jnp.where(grp == g, fg, fst)
    removed = iota == fst
    m2 = jnp.full((RB, E), NEG)
    for g in range(NG):
        mg = jnp.max(jnp.where((grp == g) & ~removed, scores, NEG),
                     axis=1, keepdims=True)
        m2 = jnp.where(grp == g, mg, m2)
    gsc = m1 + m2

    rep = iota % GS == 0
    gsc_rep = jnp.where(rep, gsc, NEG)
    g1v = jnp.max(gsc_rep, axis=1, keepdims=True)
    g1lane = jnp.min(jnp.where(gsc_rep == g1v, iota, E), axis=1, keepdims=True)
    gsc_rep2 = jnp.where(iota == g1lane, NEG, gsc_rep)
    g2v = jnp.max(gsc_rep2, axis=1, keepdims=True)
    g2lane = jnp.min(jnp.where(gsc_rep2 == g2v, iota, E), axis=1, keepdims=True)
    gmask = (grp == g1lane // GS) | (grp == g2lane // GS)

    masked = jnp.where(gmask, scores, NEG)
    v1 = jnp.max(masked, axis=1, keepdims=True)
    e1 = jnp.min(jnp.where(masked == v1, iota, E), axis=1, keepdims=True)
    masked2 = jnp.where(iota == e1, NEG, masked)
    v2 = jnp.max(masked2, axis=1, keepdims=True)
    e2 = jnp.min(jnp.where(masked2 == v2, iota, E), axis=1, keepdims=True)
    denom = v1 + v2 + 1e-20
    iota2 = lax.broadcasted_iota(I32, (RB, K), 1)
    logits_ref[...] = logits
    tidx_ref[...] = jnp.where(iota2 == 0, e1, e2)
    tw_ref[...] = jnp.where(iota2 == 0, v1 / denom, v2 / denom)
    xbf_ref[...] = _pack_row(x[:, :HW], x[:, HW:])


def _route(flat, gW):
    return pl.pallas_call(
        _routing_body,
        grid=(T // RB,),
        in_specs=[pl.BlockSpec((RB, H), lambda b: (b, 0)),
                  pl.BlockSpec((E, H), lambda b: (0, 0))],
        out_specs=[pl.BlockSpec((RB, E), lambda b: (b, 0)),
                   pl.BlockSpec((RB, K), lambda b: (b, 0)),
                   pl.BlockSpec((RB, K), lambda b: (b, 0)),
                   pl.BlockSpec((RB, HW), lambda b: (b, 0))],
        out_shape=[jax.ShapeDtypeStruct((T, E), F32),
                   jax.ShapeDtypeStruct((T, K), I32),
                   jax.ShapeDtypeStruct((T, K), F32),
                   jax.ShapeDtypeStruct((T, HW), I32)],
    )(flat, gW)


def _dispatch_body(e_ref, pos_ref, be_ref, acc, bs, po, tot, tril_s):
    s = pl.program_id(0)
    b = pl.program_id(1)
    e = e_ref[...]
    iota = lax.broadcasted_iota(I32, (PB, E), 1)
    oh = jnp.where(e == iota, 1.0, 0.0).astype(F32)

    @pl.when((s == 0) & (b == 0))
    def _():
        acc[...] = jnp.zeros_like(acc)
        r = lax.broadcasted_iota(I32, (PB, PB), 0)
        c = lax.broadcasted_iota(I32, (PB, PB), 1)
        tril_s[...] = jnp.where(r > c, 1.0, 0.0)

    @pl.when(s == 0)
    def _():
        bs[pl.ds(b, 1), :] = acc[0:1, :]
        acc[0:1, :] = acc[0:1, :] + jnp.sum(oh, axis=0, keepdims=True)

    @pl.when((s == 1) & (b == 0))
    def _():
        ci = acc[0:1, :].astype(I32)
        pc = ((ci + (BM - 1)) >> BMLOG) << BMLOG
        pcf = pc.astype(F32)
        r = lax.broadcasted_iota(I32, (E, E), 0)
        c = lax.broadcasted_iota(I32, (E, E), 1)
        m = jnp.where(r < c, 1.0, 0.0)
        po[0:1, :] = lax.dot_general(pcf, m, (((1,), (0,)), ((), ())),
                                     preferred_element_type=F32,
                                     precision=lax.Precision.HIGHEST)
        tot[0:1, :] = jnp.broadcast_to(jnp.sum(pcf, axis=1, keepdims=True),
                                       (1, E))

    @pl.when(s == 1)
    def _():
        ranks = lax.dot_general(tril_s[...], oh, (((1,), (0,)), ((), ())),
                                preferred_element_type=F32,
                                precision=lax.Precision.HIGHEST)
        rank = jnp.sum(ranks * oh, axis=1, keepdims=True)
        base = jnp.sum(oh * (po[0:1, :] + bs[pl.ds(b, 1), :]),
                       axis=1, keepdims=True)
        pos_ref[...] = (rank + base).astype(I32)
        bi = lax.broadcasted_iota(I32, (NBLK, E), 0).astype(F32) * BM
        nle = jnp.sum(jnp.where(bi >= po[0:1, :], 1.0, 0.0),
                      axis=1, keepdims=True)
        bstart = lax.broadcasted_iota(I32, (NBLK, 1), 0).astype(F32) * BM
        be_ref[...] = jnp.where(bstart < tot[0:1, 0:1],
                                nle - 1.0, -1.0).astype(I32)


def _dispatch(e_col):
    return pl.pallas_call(
        _dispatch_body,
        grid=(2, NPB),
        in_specs=[pl.BlockSpec((PB, 1), lambda s, b: (b, 0))],
        out_specs=[pl.BlockSpec((PB, 1), lambda s, b: (b, 0)),
                   pl.BlockSpec((NBLK, 1), lambda s, b: (0, 0))],
        out_shape=[jax.ShapeDtypeStruct((P, 1), I32),
                   jax.ShapeDtypeStruct((NBLK, 1), I32)],
        scratch_shapes=[pltpu.VMEM((8, E), F32),
                        pltpu.VMEM((NPB, E), F32),
                        pltpu.VMEM((8, E), F32),
                        pltpu.VMEM((8, E), F32),
                        pltpu.VMEM((PB, PB), F32)],
    )(e_col)


_C3 = 4


@functools.cache
def _sc_mesh():
    return plsc.VectorSubcoreMesh(core_axis_name="c", subcore_axis_name="s",
                                  num_cores=2, num_subcores=16)


@functools.cache
def _build_scatter():
    @functools.partial(
        pl.kernel, mesh=_sc_mesh(),
        out_type=jax.ShapeDtypeStruct((MEXP, HW), I32),
        scratch_types=[pltpu.VMEM((_C3, 32), I32),
                       pltpu.VMEM((_C3, 32), I32),
                       pltpu.VMEM((32, HW), I32),
                       pltpu.SemaphoreType.DMA],
    )
    def scatter_kernel(flat_hbm, pe_hbm, po_hbm, xs_hbm,
                       idxe_v, idxo_v, rows_v, sem):
        wid = lax.axis_index("s") * 2 + lax.axis_index("c")
        pltpu.sync_copy(pe_hbm.at[wid], idxe_v)
        pltpu.sync_copy(po_hbm.at[wid], idxo_v)
        for c in range(_C3):
            base = wid * (T // NW) + c * 32
            pltpu.sync_copy(flat_hbm.at[pl.ds(base, 32)], rows_v)
            pltpu.async_copy(rows_v, xs_hbm.at[idxe_v.at[c]], sem).wait()
            pltpu.async_copy(rows_v, xs_hbm.at[idxo_v.at[c]], sem).wait()

    return scatter_kernel


def _scatter(flat, pe3, po3):
    return _build_scatter()(flat, pe3, po3)


def _mlp_packed(xw, gw, uw, dw):
    xlo_f, xhi_f = _unpack_row(xw)
    xlo = xlo_f.astype(jnp.bfloat16)
    xhi = xhi_f.astype(jnp.bfloat16)
    dn = (((1,), (1,)), ((), ()))
    gwb = gw.astype(jnp.bfloat16)
    uwb = uw.astype(jnp.bfloat16)
    g = (lax.dot_general(xlo, gwb[:, :HW], dn, preferred_element_type=F32)
         + lax.dot_general(xhi, gwb[:, HW:], dn, preferred_element_type=F32))
    u = (lax.dot_general(xlo, uwb[:, :HW], dn, preferred_element_type=F32)
         + lax.dot_general(xhi, uwb[:, HW:], dn, preferred_element_type=F32))
    h = (g / (1.0 + jnp.exp(-g)) * u).astype(jnp.bfloat16)
    y = lax.dot_general(h, dw.astype(jnp.bfloat16), dn,
                        preferred_element_type=F32)
    return _pack_row(y[:, :HW], y[:, HW:])


def _gmm_body(be_ref, x_ref, gw_ref, uw_ref, dw_ref, o_ref):
    b = pl.program_id(0)

    @pl.when(be_ref[b] >= 0)
    def _():
        o_ref[...] = _mlp_packed(x_ref[...], gw_ref[0], uw_ref[0], dw_ref[0])


def _gmm(be, xs, egw, euw, edw):
    grid_spec = pltpu.PrefetchScalarGridSpec(
        num_scalar_prefetch=1,
        grid=(NBLK,),
        in_specs=[
            pl.BlockSpec((BM, HW), lambda b, be_ref: (b, 0)),
            pl.BlockSpec((1, FF, H),
                         lambda b, be_ref: (jnp.maximum(be_ref[b], 0), 0, 0)),
            pl.BlockSpec((1, FF, H),
                         lambda b, be_ref: (jnp.maximum(be_ref[b], 0), 0, 0)),
            pl.BlockSpec((1, H, FF),
                         lambda b, be_ref: (jnp.maximum(be_ref[b], 0), 0, 0)),
        ],
        out_specs=pl.BlockSpec((BM, HW), lambda b, be_ref: (b, 0)),
    )
    return pl.pallas_call(
        _gmm_body, grid_spec=grid_spec,
        out_shape=jax.ShapeDtypeStruct((MEXP, HW), I32),
    )(be, xs, egw, euw, edw)


def _shared_body(x_ref, gw_ref, uw_ref, dw_ref, o_ref):
    o_ref[...] = _mlp_packed(x_ref[...], gw_ref[...], uw_ref[...],
                             dw_ref[...])


def _shared(flat_packed, sgw, suw, sdw):
    return pl.pallas_call(
        _shared_body,
        grid=(T // BM,),
        in_specs=[pl.BlockSpec((BM, HW), lambda b: (b, 0)),
                  pl.BlockSpec((FF, H), lambda b: (0, 0)),
                  pl.BlockSpec((FF, H), lambda b: (0, 0)),
                  pl.BlockSpec((H, FF), lambda b: (0, 0))],
        out_specs=pl.BlockSpec((BM, HW), lambda b: (b, 0)),
        out_shape=jax.ShapeDtypeStruct((T, HW), I32),
    )(flat_packed, sgw, suw, sdw)


_CT = 8
_C5 = (T // NW) // _CT


@functools.cache
def _build_combine():
    @functools.partial(
        pl.kernel, mesh=_sc_mesh(),
        out_type=jax.ShapeDtypeStruct((T, H), F32),
        scratch_types=[pltpu.VMEM((_C5, _CT), I32),
                       pltpu.VMEM((_C5, _CT), I32),
                       pltpu.VMEM((_C5 * _CT // 16, 16), F32),
                       pltpu.VMEM((_C5 * _CT // 16, 16), F32),
                       pltpu.VMEM((2, _CT, HW), I32),
                       pltpu.VMEM((2, _CT, HW), I32),
                       pltpu.VMEM((2, _CT, HW), I32),
                       pltpu.VMEM((_CT, H), F32),
                       pltpu.SemaphoreType.DMA,
                       pltpu.SemaphoreType.DMA],
    )
    def combine_kernel(ys_hbm, ysh_hbm, pe_hbm, po_hbm, we_hbm, wo_hbm, y_hbm,
                       idxe_v, idxo_v, we_v, wo_v, r0_v, r1_v, rsh_v, out_v,
                       sem0, sem1):
        wid = lax.axis_index("s") * 2 + lax.axis_index("c")
        pltpu.sync_copy(pe_hbm.at[wid], idxe_v)
        pltpu.sync_copy(po_hbm.at[wid], idxo_v)
        pltpu.sync_copy(we_hbm.at[wid], we_v)
        pltpu.sync_copy(wo_hbm.at[wid], wo_v)
        sems = (sem0, sem1)

        def issue(c, slot):
            base = wid * (T // NW) + c * _CT
            return (
                pltpu.async_copy(ys_hbm.at[idxe_v.at[c]], r0_v.at[slot],
                                 sems[slot]),
                pltpu.async_copy(ys_hbm.at[idxo_v.at[c]], r1_v.at[slot],
                                 sems[slot]),
                pltpu.async_copy(ysh_hbm.at[pl.ds(base, _CT)], rsh_v.at[slot],
                                 sems[slot]),
            )

        handles = [issue(0, 0), None]
        for c in range(_C5):
            slot = c % 2
            if c + 1 < _C5:
                handles[1 - slot] = issue(c + 1, 1 - slot)
            for h in handles[slot]:
                h.wait()
            base = wid * (T // NW) + c * _CT
            w0r = we_v[c // 2, :]
            w1r = wo_v[c // 2, :]
            off = (c % 2) * _CT

            def tok(i, _):
                sel = jnp.zeros((16,), I32) + (off + i)
                b0 = jnp.take(w0r, sel)
                b1 = jnp.take(w1r, sel)

                def col(j, _):
                    sl = pl.ds(j * 16, 16)
                    r0lo, r0hi = _unpack_row(r0_v[slot, i, sl])
                    r1lo, r1hi = _unpack_row(r1_v[slot, i, sl])
                    shlo, shhi = _unpack_row(rsh_v[slot, i, sl])
                    out_v[i, sl] = b0 * r0lo + b1 * r1lo + shlo
                    hi_sl = pl.ds(HW + j * 16, 16)
                    out_v[i, hi_sl] = b0 * r0hi + b1 * r1hi + shhi
                    return 0

                lax.fori_loop(0, HW // 16, col, 0, unroll=8)
                return 0

            lax.fori_loop(0, _CT, tok, 0)
            pltpu.sync_copy(out_v, y_hbm.at[pl.ds(base, _CT)])

    return combine_kernel


def _combine(ys, ysh, pe5, po5, w05, w15):
    return _build_combine()(ys, ysh, pe5, po5, w05, w15)


def kernel(hidden_states, gate_weight, expert_gate_w, expert_up_w,
           expert_down_w, shared_gate_w, shared_up_w, shared_down_w):
    flat = hidden_states.reshape(T, H)
    logits, tidx, tw, flat_bf = _route(flat, gate_weight)
    pos_col, be_col = _dispatch(tidx.reshape(P, 1))
    pos = pos_col.reshape(T, K)
    be = be_col.reshape(NBLK)
    pe3 = pos[:, 0].reshape(NW, _C3, 32)
    po3 = pos[:, 1].reshape(NW, _C3, 32)
    xs = _scatter(flat_bf, pe3, po3)
    ysh = _shared(flat_bf, shared_gate_w, shared_up_w, shared_down_w)
    ys = _gmm(be, xs, expert_gate_w, expert_up_w, expert_down_w)
    pe5 = pos[:, 0].reshape(NW, _C5, _CT)
    po5 = pos[:, 1].reshape(NW, _C5, _CT)
    w05 = tw[:, 0].reshape(NW, _C5 * _CT // 16, 16)
    w15 = tw[:, 1].reshape(NW, _C5 * _CT // 16, 16)
    y = _combine(ys, ysh, pe5, po5, w05, w15)
    return y.reshape(B, S, H), (logits.reshape(B, S, E), tidx.reshape(B, S, K))


# speedup vs baseline: 4.5138x; 1.0151x over previous
import functools

import jax
import jax.numpy as jnp
from jax import lax
from jax.experimental import pallas as pl
from jax.experimental.pallas import tpu as pltpu
from jax.experimental.pallas import tpu_sc as plsc

F32 = jnp.float32
I32 = jnp.int32
NEG = -1e30

B, S, H = 2, 2048, 1024
E, K = 16, 2
FF = 512
NG = 4
GS = E // NG
T = B * S
P = T * K
BM = 256
BMLOG = 8
NBLK = (P + E * (BM - 1) + BM - 1) // BM
MEXP = NBLK * BM
NBLKT = NBLK + T // BM
RB = 512
PB = 512
NPB = P // PB
NW = 32
HW = H // 2


def _pack_row(xlo, xhi):
    ulo = lax.bitcast_convert_type(xlo, jnp.uint32)
    uhi = lax.bitcast_convert_type(xhi, jnp.uint32)
    rlo = ulo + jnp.uint32(0x7FFF) + ((ulo >> 16) & jnp.uint32(1))
    rhi = uhi + jnp.uint32(0x7FFF) + ((uhi >> 16) & jnp.uint32(1))
    w = (rlo >> 16) | (rhi & jnp.uint32(0xFFFF0000))
    return lax.bitcast_convert_type(w, I32)


def _unpack_row(w):
    lo = lax.bitcast_convert_type(w << 16, F32)
    hi = lax.bitcast_convert_type(w & jnp.int32(-65536), F32)
    return lo, hi


def _routing_body(x_ref, gw_ref, logits_ref, tidx_ref, tw_ref, xbf_ref):
    x = x_ref[...]
    gw = gw_ref[...]
    logits = lax.dot_general(x, gw, (((1,), (1,)), ((), ())),
                             preferred_element_type=F32)
    scores = 1.0 / (1.0 + jnp.exp(-logits))
    iota = lax.broadcasted_iota(I32, (RB, E), 1)
    grp = iota // GS

    m1 = jnp.full((RB, E), NEG)
    for g in range(NG):
        mg = jnp.max(jnp.where(grp == g, scores, NEG), axis=1, keepdims=True)
        m1 = jnp.where(grp == g, mg, m1)
    cand = jnp.where(scores == m1, iota, E)
    fst = jnp.zeros((RB, E), I32)
    for g in range(NG):
        fg = jnp.min(jnp.where(grp == g, cand, E), axis=1, keepdims=True)
        fst = jnp.where(grp == g, fg, fst)
    removed = iota == fst
    m2 = jnp.full((RB, E), NEG)
    for g in range(NG):
        mg = jnp.max(jnp.where((grp == g) & ~removed, scores, NEG),
                     axis=1, keepdims=True)
        m2 = jnp.where(grp == g, mg, m2)
    gsc = m1 + m2

    rep = iota % GS == 0
    gsc_rep = jnp.where(rep, gsc, NEG)
    g1v = jnp.max(gsc_rep, axis=1, keepdims=True)
    g1lane = jnp.min(jnp.where(gsc_rep == g1v, iota, E), axis=1, keepdims=True)
    gsc_rep2 = jnp.where(iota == g1lane, NEG, gsc_rep)
    g2v = jnp.max(gsc_rep2, axis=1, keepdims=True)
    g2lane = jnp.min(jnp.where(gsc_rep2 == g2v, iota, E), axis=1, keepdims=True)
    gmask = (grp == g1lane // GS) | (grp == g2lane // GS)

    masked = jnp.where(gmask, scores, NEG)
    v1 = jnp.max(masked, axis=1, keepdims=True)
    e1 = jnp.min(jnp.where(masked == v1, iota, E), axis=1, keepdims=True)
    masked2 = jnp.where(iota == e1, NEG, masked)
    v2 = jnp.max(masked2, axis=1, keepdims=True)
    e2 = jnp.min(jnp.where(masked2 == v2, iota, E), axis=1, keepdims=True)
    denom = v1 + v2 + 1e-20
    iota2 = lax.broadcasted_iota(I32, (RB, K), 1)
    logits_ref[...] = logits
    tidx_ref[...] = jnp.where(iota2 == 0, e1, e2)
    tw_ref[...] = jnp.where(iota2 == 0, v1 / denom, v2 / denom)
    xbf_ref[...] = _pack_row(x[:, :HW], x[:, HW:])


def _route(flat, gW):
    return pl.pallas_call(
        _routing_body,
        grid=(T // RB,),
        in_specs=[pl.BlockSpec((RB, H), lambda b: (b, 0)),
                  pl.BlockSpec((E, H), lambda b: (0, 0))],
        out_specs=[pl.BlockSpec((RB, E), lambda b: (b, 0)),
                   pl.BlockSpec((RB, K), lambda b: (b, 0)),
                   pl.BlockSpec((RB, K), lambda b: (b, 0)),
                   pl.BlockSpec((RB, HW), lambda b: (b, 0))],
        out_shape=[jax.ShapeDtypeStruct((T, E), F32),
                   jax.ShapeDtypeStruct((T, K), I32),
                   jax.ShapeDtypeStruct((T, K), F32),
                   jax.ShapeDtypeStruct((T, HW), I32)],
    )(flat, gW)


def _dispatch_body(e_ref, pos_ref, be_ref, acc, bs, po, tot, tril_s):
    s = pl.program_id(0)
    b = pl.program_id(1)
    e = e_ref[...]
    iota = lax.broadcasted_iota(I32, (PB, E), 1)
    oh = jnp.where(e == iota, 1.0, 0.0).astype(F32)

    @pl.when((s == 0) & (b == 0))
    def _():
        acc[...] = jnp.zeros_like(acc)
        r = lax.broadcasted_iota(I32, (PB, PB), 0)
        c = lax.broadcasted_iota(I32, (PB, PB), 1)
        tril_s[...] = jnp.where(r > c, 1.0, 0.0).astype(jnp.bfloat16)

    @pl.when(s == 0)
    def _():
        bs[pl.ds(b, 1), :] = acc[0:1, :]
        acc[0:1, :] = acc[0:1, :] + jnp.sum(oh, axis=0, keepdims=True)

    @pl.when((s == 1) & (b == 0))
    def _():
        ci = acc[0:1, :].astype(I32)
        pc = ((ci + (BM - 1)) >> BMLOG) << BMLOG
        pcf = pc.astype(F32)
        r = lax.broadcasted_iota(I32, (E, E), 0)
        c = lax.broadcasted_iota(I32, (E, E), 1)
        m = jnp.where(r < c, 1.0, 0.0)
        po[0:1, :] = lax.dot_general(pcf, m, (((1,), (0,)), ((), ())),
                                     preferred_element_type=F32,
                                     precision=lax.Precision.HIGHEST)
        tot[0:1, :] = jnp.broadcast_to(jnp.sum(pcf, axis=1, keepdims=True),
                                       (1, E))

    @pl.when(s == 1)
    def _():
        ranks = lax.dot_general(tril_s[...], oh.astype(jnp.bfloat16),
                                (((1,), (0,)), ((), ())),
                                preferred_element_type=F32)
        rank = jnp.sum(ranks * oh, axis=1, keepdims=True)
        base = jnp.sum(oh * (po[0:1, :] + bs[pl.ds(b, 1), :]),
                       axis=1, keepdims=True)
        pos_ref[...] = (rank + base).astype(I32)
        bi = lax.broadcasted_iota(I32, (NBLKT, E), 0).astype(F32) * BM
        nle = jnp.sum(jnp.where(bi >= po[0:1, :], 1.0, 0.0),
                      axis=1, keepdims=True)
        bstart = lax.broadcasted_iota(I32, (NBLKT, 1), 0).astype(F32) * BM
        rowi = lax.broadcasted_iota(I32, (NBLKT, 1), 0)
        be_ref[...] = jnp.where(
            rowi >= NBLK, E,
            jnp.where(bstart < tot[0:1, 0:1], nle - 1.0, -1.0).astype(I32))


def _dispatch(e_col):
    return pl.pallas_call(
        _dispatch_body,
        grid=(2, NPB),
        in_specs=[pl.BlockSpec((PB, 1), lambda s, b: (b, 0))],
        out_specs=[pl.BlockSpec((PB, 1), lambda s, b: (b, 0)),
                   pl.BlockSpec((NBLKT, 1), lambda s, b: (0, 0))],
        out_shape=[jax.ShapeDtypeStruct((P, 1), I32),
                   jax.ShapeDtypeStruct((NBLKT, 1), I32)],
        scratch_shapes=[pltpu.VMEM((8, E), F32),
                        pltpu.VMEM((NPB, E), F32),
                        pltpu.VMEM((8, E), F32),
                        pltpu.VMEM((8, E), F32),
                        pltpu.VMEM((PB, PB), jnp.bfloat16)],
    )(e_col)


_C3 = 4


@functools.cache
def _sc_mesh():
    return plsc.VectorSubcoreMesh(core_axis_name="c", subcore_axis_name="s",
                                  num_cores=2, num_subcores=16)


@functools.cache
def _build_scatter():
    @functools.partial(
        pl.kernel, mesh=_sc_mesh(),
        out_type=jax.ShapeDtypeStruct((MEXP, HW), I32),
        scratch_types=[pltpu.VMEM((_C3, 32), I32),
                       pltpu.VMEM((_C3, 32), I32),
                       pltpu.VMEM((32, HW), I32),
                       pltpu.SemaphoreType.DMA],
    )
    def scatter_kernel(flat_hbm, pe_hbm, po_hbm, xs_hbm,
                       idxe_v, idxo_v, rows_v, sem):
        wid = lax.axis_index("s") * 2 + lax.axis_index("c")
        pltpu.sync_copy(pe_hbm.at[wid], idxe_v)
        pltpu.sync_copy(po_hbm.at[wid], idxo_v)
        for c in range(_C3):
            base = wid * (T // NW) + c * 32
            pltpu.sync_copy(flat_hbm.at[pl.ds(base, 32)], rows_v)
            pltpu.async_copy(rows_v, xs_hbm.at[idxe_v.at[c]], sem).wait()
            pltpu.async_copy(rows_v, xs_hbm.at[idxo_v.at[c]], sem).wait()

    return scatter_kernel


def _scatter(flat, pe3, po3):
    return _build_scatter()(flat, pe3, po3)


def _mlp_packed(xw, gw, uw, dw):
    xlo_f, xhi_f = _unpack_row(xw)
    xlo = xlo_f.astype(jnp.bfloat16)
    xhi = xhi_f.astype(jnp.bfloat16)
    dn = (((1,), (1,)), ((), ()))
    gwb = gw.astype(jnp.bfloat16)
    uwb = uw.astype(jnp.bfloat16)
    g = (lax.dot_general(xlo, gwb[:, :HW], dn, preferred_element_type=F32)
         + lax.dot_general(xhi, gwb[:, HW:], dn, preferred_element_type=F32))
    u = (lax.dot_general(xlo, uwb[:, :HW], dn, preferred_element_type=F32)
         + lax.dot_general(xhi, uwb[:, HW:], dn, preferred_element_type=F32))
    h = (g / (1.0 + jnp.exp(-g)) * u).astype(jnp.bfloat16)
    y = lax.dot_general(h, dw.astype(jnp.bfloat16), dn,
                        preferred_element_type=F32)
    return _pack_row(y[:, :HW], y[:, HW:])


def _gmm_body(be_ref, x_ref, fl_ref, gw_ref, uw_ref, dw_ref,
              sg_ref, su_ref, sd_ref, o_ref, osh_ref):
    b = pl.program_id(0)
    e = be_ref[b]

    @pl.when((e >= 0) & (e < E))
    def _():
        o_ref[...] = _mlp_packed(x_ref[...], gw_ref[0], uw_ref[0], dw_ref[0])

    @pl.when(e == E)
    def _():
        osh_ref[...] = _mlp_packed(fl_ref[...], sg_ref[...], su_ref[...],
                                   sd_ref[...])


def _gmm(be, xs, flat_packed, egw, euw, edw, sgw, suw, sdw):
    def wmap(b, be_ref):
        return (jnp.clip(be_ref[b], 0, E - 1), 0, 0)

    grid_spec = pltpu.PrefetchScalarGridSpec(
        num_scalar_prefetch=1,
        grid=(NBLKT,),
        in_specs=[
            pl.BlockSpec((BM, HW),
                         lambda b, be_ref: (jnp.minimum(b, NBLK - 1), 0)),
            pl.BlockSpec((BM, HW),
                         lambda b, be_ref: (jnp.maximum(b - NBLK, 0), 0)),
            pl.BlockSpec((1, FF, H), wmap),
            pl.BlockSpec((1, FF, H), wmap),
            pl.BlockSpec((1, H, FF), wmap),
            pl.BlockSpec((FF, H), lambda b, be_ref: (0, 0)),
            pl.BlockSpec((FF, H), lambda b, be_ref: (0, 0)),
            pl.BlockSpec((H, FF), lambda b, be_ref: (0, 0)),
        ],
        out_specs=[
            pl.BlockSpec((BM, HW),
                         lambda b, be_ref: (jnp.minimum(b, NBLK - 1), 0)),
            pl.BlockSpec((BM, HW),
                         lambda b, be_ref: (jnp.maximum(b - NBLK, 0), 0)),
        ],
    )
    return pl.pallas_call(
        _gmm_body, grid_spec=grid_spec,
        out_shape=[jax.ShapeDtypeStruct((MEXP, HW), I32),
                   jax.ShapeDtypeStruct((T, HW), I32)],
    )(be, xs, flat_packed, egw, euw, edw, sgw, suw, sdw)


_CT = 8
_C5 = (T // NW) // _CT


@functools.cache
def _build_combine():
    @functools.partial(
        pl.kernel, mesh=_sc_mesh(),
        out_type=jax.ShapeDtypeStruct((T, H), F32),
        scratch_types=[pltpu.VMEM((_C5, _CT), I32),
                       pltpu.VMEM((_C5, _CT), I32),
                       pltpu.VMEM((_C5 * _CT // 16, 16), F32),
                       pltpu.VMEM((_C5 * _CT // 16, 16), F32),
                       pltpu.VMEM((2, _CT, HW), I32),
                       pltpu.VMEM((2, _CT, HW), I32),
                       pltpu.VMEM((2, _CT, HW), I32),
                       pltpu.VMEM((2, _CT, H), F32),
                       pltpu.SemaphoreType.DMA,
                       pltpu.SemaphoreType.DMA,
                       pltpu.SemaphoreType.DMA],
    )
    def combine_kernel(ys_hbm, ysh_hbm, pe_hbm, po_hbm, we_hbm, wo_hbm, y_hbm,
                       idxe_v, idxo_v, we_v, wo_v, r0_v, r1_v, rsh_v, out_v,
                       sem0, sem1, sem_out):
        wid = lax.axis_index("s") * 2 + lax.axis_index("c")
        pltpu.sync_copy(pe_hbm.at[wid], idxe_v)
        pltpu.sync_copy(po_hbm.at[wid], idxo_v)
        pltpu.sync_copy(we_hbm.at[wid], we_v)
        pltpu.sync_copy(wo_hbm.at[wid], wo_v)
        sems = (sem0, sem1)

        def issue(c, slot):
            base = wid * (T // NW) + c * _CT
            return (
                pltpu.async_copy(ys_hbm.at[idxe_v.at[c]], r0_v.at[slot],
                                 sems[slot]),
                pltpu.async_copy(ys_hbm.at[idxo_v.at[c]], r1_v.at[slot],
                                 sems[slot]),
                pltpu.async_copy(ysh_hbm.at[pl.ds(base, _CT)], rsh_v.at[slot],
                                 sems[slot]),
            )

        handles = [issue(0, 0), None]
        out_handles = [None, None]
        for c in range(_C5):
            slot = c % 2
            if c + 1 < _C5:
                handles[1 - slot] = issue(c + 1, 1 - slot)
            for h in handles[slot]:
                h.wait()
            if out_handles[slot] is not None:
                out_handles[slot].wait()
            base = wid * (T // NW) + c * _CT
            w0r = we_v[c // 2, :]
            w1r = wo_v[c // 2, :]
            off = (c % 2) * _CT

            def tok(i, _):
                sel = jnp.zeros((16,), I32) + (off + i)
                b0 = jnp.take(w0r, sel)
                b1 = jnp.take(w1r, sel)

                def col(j, _):
                    sl = pl.ds(j * 16, 16)
                    r0lo, r0hi = _unpack_row(r0_v[slot, i, sl])
                    r1lo, r1hi = _unpack_row(r1_v[slot, i, sl])
                    shlo, shhi = _unpack_row(rsh_v[slot, i, sl])
                    out_v[slot, i, sl] = b0 * r0lo + b1 * r1lo + shlo
                    hi_sl = pl.ds(HW + j * 16, 16)
                    out_v[slot, i, hi_sl] = b0 * r0hi + b1 * r1hi + shhi
                    return 0

                lax.fori_loop(0, HW // 16, col, 0, unroll=8)
                return 0

            lax.fori_loop(0, _CT, tok, 0)
            out_handles[slot] = pltpu.async_copy(
                out_v.at[slot], y_hbm.at[pl.ds(base, _CT)], sem_out)
        for oh_ in out_handles:
            if oh_ is not None:
                oh_.wait()

    return combine_kernel


def _combine(ys, ysh, pe5, po5, w05, w15):
    return _build_combine()(ys, ysh, pe5, po5, w05, w15)


def kernel(hidden_states, gate_weight, expert_gate_w, expert_up_w,
           expert_down_w, shared_gate_w, shared_up_w, shared_down_w):
    flat = hidden_states.reshape(T, H)
    logits, tidx, tw, flat_bf = _route(flat, gate_weight)
    pos_col, be_col = _dispatch(tidx.reshape(P, 1))
    pos = pos_col.reshape(T, K)
    be = be_col.reshape(NBLKT)
    pe3 = pos[:, 0].reshape(NW, _C3, 32)
    po3 = pos[:, 1].reshape(NW, _C3, 32)
    xs = _scatter(flat_bf, pe3, po3)
    ys, ysh = _gmm(be, xs, flat_bf, expert_gate_w, expert_up_w, expert_down_w,
                   shared_gate_w, shared_up_w, shared_down_w)
    pe5 = pos[:, 0].reshape(NW, _C5, _CT)
    po5 = pos[:, 1].reshape(NW, _C5, _CT)
    w05 = tw[:, 0].reshape(NW, _C5 * _CT // 16, 16)
    w15 = tw[:, 1].reshape(NW, _C5 * _CT // 16, 16)
    y = _combine(ys, ysh, pe5, po5, w05, w15)
    return y.reshape(B, S, H), (logits.reshape(B, S, E), tidx.reshape(B, S, K))


# speedup vs baseline: 4.8061x; 1.0648x over previous
import functools

import jax
import jax.numpy as jnp
from jax import lax
from jax.experimental import pallas as pl
from jax.experimental.pallas import tpu as pltpu
from jax.experimental.pallas import tpu_sc as plsc

F32 = jnp.float32
I32 = jnp.int32
NEG = -1e30

B, S, H = 2, 2048, 1024
E, K = 16, 2
FF = 512
NG = 4
GS = E // NG
T = B * S
P = T * K
BM = 256
BMLOG = 8
NBLK = (P + E * (BM - 1) + BM - 1) // BM
MEXP = NBLK * BM
NBLKT = NBLK + T // BM
RB = 512
PB = 512
NPB = P // PB
NW = 32
HW = H // 2


def _pack_row(xlo, xhi):
    ulo = lax.bitcast_convert_type(xlo, jnp.uint32)
    uhi = lax.bitcast_convert_type(xhi, jnp.uint32)
    rlo = ulo + jnp.uint32(0x7FFF) + ((ulo >> 16) & jnp.uint32(1))
    rhi = uhi + jnp.uint32(0x7FFF) + ((uhi >> 16) & jnp.uint32(1))
    w = (rlo >> 16) | (rhi & jnp.uint32(0xFFFF0000))
    return lax.bitcast_convert_type(w, I32)


def _unpack_row(w):
    lo = lax.bitcast_convert_type(w << 16, F32)
    hi = lax.bitcast_convert_type(w & jnp.int32(-65536), F32)
    return lo, hi


def _routing_body(x_ref, gw_ref, logits_ref, tidx_ref, tw_ref, xbf_ref):
    x = x_ref[...]
    gw = gw_ref[...]
    logits = lax.dot_general(x, gw, (((1,), (1,)), ((), ())),
                             preferred_element_type=F32)
    lt = jnp.transpose(logits)
    logits_ref[...] = lt
    xbf_ref[...] = _pack_row(x[:, :HW], x[:, HW:])

    scores = 1.0 / (1.0 + jnp.exp(-lt))
    iota = lax.broadcasted_iota(I32, (E, RB), 0)
    grp = iota // GS

    m1 = jnp.full((E, RB), NEG)
    for g in range(NG):
        mg = jnp.max(scores[g * GS:(g + 1) * GS], axis=0, keepdims=True)
        m1 = jnp.where(grp == g, mg, m1)
    cand = jnp.where(scores == m1, iota, E)
    fst = jnp.zeros((E, RB), I32)
    for g in range(NG):
        fg = jnp.min(cand[g * GS:(g + 1) * GS], axis=0, keepdims=True)
        fst = jnp.where(grp == g, fg, fst)
    removed = iota == fst
    scores2 = jnp.where(removed, NEG, scores)
    m2 = jnp.full((E, RB), NEG)
    for g in range(NG):
        mg = jnp.max(scores2[g * GS:(g + 1) * GS], axis=0, keepdims=True)
        m2 = jnp.where(grp == g, mg, m2)
    gsc = m1 + m2

    rep = iota % GS == 0
    gsc_rep = jnp.where(rep, gsc, NEG)
    g1v = jnp.max(gsc_rep, axis=0, keepdims=True)
    g1row = jnp.min(jnp.where(gsc_rep == g1v, iota, E), axis=0, keepdims=True)
    gsc_rep2 = jnp.where(iota == g1row, NEG, gsc_rep)
    g2v = jnp.max(gsc_rep2, axis=0, keepdims=True)
    g2row = jnp.min(jnp.where(gsc_rep2 == g2v, iota, E), axis=0, keepdims=True)
    gmask = (grp == g1row // GS) | (grp == g2row // GS)

    masked = jnp.where(gmask, scores, NEG)
    v1 = jnp.max(masked, axis=0, keepdims=True)
    e1 = jnp.min(jnp.where(masked == v1, iota, E), axis=0, keepdims=True)
    masked2 = jnp.where(iota == e1, NEG, masked)
    v2 = jnp.max(masked2, axis=0, keepdims=True)
    e2 = jnp.min(jnp.where(masked2 == v2, iota, E), axis=0, keepdims=True)
    denom = v1 + v2 + 1e-20

    rowi = lax.broadcasted_iota(I32, (8, RB), 0)
    tidx_ref[...] = jnp.where(rowi == 0, e1, jnp.where(rowi == 1, e2, 0))
    tw_ref[...] = jnp.where(rowi == 0, v1 / denom,
                            jnp.where(rowi == 1, v2 / denom, 0.0))


def _route(flat, gW):
    return pl.pallas_call(
        _routing_body,
        grid=(T // RB,),
        in_specs=[pl.BlockSpec((RB, H), lambda b: (b, 0)),
                  pl.BlockSpec((E, H), lambda b: (0, 0))],
        out_specs=[pl.BlockSpec((E, RB), lambda b: (0, b)),
                   pl.BlockSpec((8, RB), lambda b: (0, b)),
                   pl.BlockSpec((8, RB), lambda b: (0, b)),
                   pl.BlockSpec((RB, HW), lambda b: (b, 0))],
        out_shape=[jax.ShapeDtypeStruct((E, T), F32),
                   jax.ShapeDtypeStruct((8, T), I32),
                   jax.ShapeDtypeStruct((8, T), F32),
                   jax.ShapeDtypeStruct((T, HW), I32)],
    )(flat, gW)


def _dispatch_body(e_ref, pos_ref, be_ref, acc, bs, po, tot, tril_s):
    s = pl.program_id(0)
    b = pl.program_id(1)
    e = e_ref[...]
    iota = lax.broadcasted_iota(I32, (PB, E), 1)
    oh = jnp.where(e == iota, 1.0, 0.0).astype(F32)

    @pl.when((s == 0) & (b == 0))
    def _():
        acc[...] = jnp.zeros_like(acc)
        r = lax.broadcasted_iota(I32, (PB, PB), 0)
        c = lax.broadcasted_iota(I32, (PB, PB), 1)
        tril_s[...] = jnp.where(r > c, 1.0, 0.0).astype(jnp.bfloat16)

    @pl.when(s == 0)
    def _():
        bs[pl.ds(b, 1), :] = acc[0:1, :]
        acc[0:1, :] = acc[0:1, :] + jnp.sum(oh, axis=0, keepdims=True)

    @pl.when((s == 1) & (b == 0))
    def _():
        ci = acc[0:1, :].astype(I32)
        pc = ((ci + (BM - 1)) >> BMLOG) << BMLOG
        pcf = pc.astype(F32)
        r = lax.broadcasted_iota(I32, (E, E), 0)
        c = lax.broadcasted_iota(I32, (E, E), 1)
        m = jnp.where(r < c, 1.0, 0.0)
        po[0:1, :] = lax.dot_general(pcf, m, (((1,), (0,)), ((), ())),
                                     preferred_element_type=F32,
                                     precision=lax.Precision.HIGHEST)
        tot[0:1, :] = jnp.broadcast_to(jnp.sum(pcf, axis=1, keepdims=True),
                                       (1, E))

    @pl.when(s == 1)
    def _():
        ranks = lax.dot_general(tril_s[...], oh.astype(jnp.bfloat16),
                                (((1,), (0,)), ((), ())),
                                preferred_element_type=F32)
        rank = jnp.sum(ranks * oh, axis=1, keepdims=True)
        base = jnp.sum(oh * (po[0:1, :] + bs[pl.ds(b, 1), :]),
                       axis=1, keepdims=True)
        pos_ref[...] = (rank + base).astype(I32)
        bi = lax.broadcasted_iota(I32, (NBLKT, E), 0).astype(F32) * BM
        nle = jnp.sum(jnp.where(bi >= po[0:1, :], 1.0, 0.0),
                      axis=1, keepdims=True)
        bstart = lax.broadcasted_iota(I32, (NBLKT, 1), 0).astype(F32) * BM
        rowi = lax.broadcasted_iota(I32, (NBLKT, 1), 0)
        be_ref[...] = jnp.where(
            rowi >= NBLK, E,
            jnp.where(bstart < tot[0:1, 0:1], nle - 1.0, -1.0).astype(I32))


def _dispatch(e_col):
    return pl.pallas_call(
        _dispatch_body,
        grid=(2, NPB),
        in_specs=[pl.BlockSpec((PB, 1), lambda s, b: (b, 0))],
        out_specs=[pl.BlockSpec((PB, 1), lambda s, b: (b, 0)),
                   pl.BlockSpec((NBLKT, 1), lambda s, b: (0, 0))],
        out_shape=[jax.ShapeDtypeStruct((P, 1), I32),
                   jax.ShapeDtypeStruct((NBLKT, 1), I32)],
        scratch_shapes=[pltpu.VMEM((8, E), F32),
                        pltpu.VMEM((NPB, E), F32),
                        pltpu.VMEM((8, E), F32),
                        pltpu.VMEM((8, E), F32),
                        pltpu.VMEM((PB, PB), jnp.bfloat16)],
    )(e_col)


_C3 = 4


@functools.cache
def _sc_mesh():
    return plsc.VectorSubcoreMesh(core_axis_name="c", subcore_axis_name="s",
                                  num_cores=2, num_subcores=16)


@functools.cache
def _build_scatter():
    @functools.partial(
        pl.kernel, mesh=_sc_mesh(),
        out_type=jax.ShapeDtypeStruct((MEXP, HW), I32),
        scratch_types=[pltpu.VMEM((_C3, 32), I32),
                       pltpu.VMEM((_C3, 32), I32),
                       pltpu.VMEM((32, HW), I32),
                       pltpu.SemaphoreType.DMA],
    )
    def scatter_kernel(flat_hbm, pe_hbm, po_hbm, xs_hbm,
                       idxe_v, idxo_v, rows_v, sem):
        wid = lax.axis_index("s") * 2 + lax.axis_index("c")
        pltpu.sync_copy(pe_hbm.at[wid], idxe_v)
        pltpu.sync_copy(po_hbm.at[wid], idxo_v)
        for c in range(_C3):
            base = wid * (T // NW) + c * 32
            pltpu.sync_copy(flat_hbm.at[pl.ds(base, 32)], rows_v)
            pltpu.async_copy(rows_v, xs_hbm.at[idxe_v.at[c]], sem).wait()
            pltpu.async_copy(rows_v, xs_hbm.at[idxo_v.at[c]], sem).wait()

    return scatter_kernel


def _scatter(flat, pe3, po3):
    return _build_scatter()(flat, pe3, po3)


def _mlp_packed(xw, gw, uw, dw):
    xlo_f, xhi_f = _unpack_row(xw)
    xlo = xlo_f.astype(jnp.bfloat16)
    xhi = xhi_f.astype(jnp.bfloat16)
    dn = (((1,), (1,)), ((), ()))
    gwb = gw.astype(jnp.bfloat16)
    uwb = uw.astype(jnp.bfloat16)
    g = (lax.dot_general(xlo, gwb[:, :HW], dn, preferred_element_type=F32)
         + lax.dot_general(xhi, gwb[:, HW:], dn, preferred_element_type=F32))
    u = (lax.dot_general(xlo, uwb[:, :HW], dn, preferred_element_type=F32)
         + lax.dot_general(xhi, uwb[:, HW:], dn, preferred_element_type=F32))
    h = (g / (1.0 + jnp.exp(-g)) * u).astype(jnp.bfloat16)
    y = lax.dot_general(h, dw.astype(jnp.bfloat16), dn,
                        preferred_element_type=F32)
    return _pack_row(y[:, :HW], y[:, HW:])


def _gmm_body(be_ref, x_ref, fl_ref, gw_ref, uw_ref, dw_ref,
              sg_ref, su_ref, sd_ref, o_ref, osh_ref):
    b = pl.program_id(0)
    e = be_ref[b]

    @pl.when((e >= 0) & (e < E))
    def _():
        o_ref[...] = _mlp_packed(x_ref[...], gw_ref[0], uw_ref[0], dw_ref[0])

    @pl.when(e == E)
    def _():
        osh_ref[...] = _mlp_packed(fl_ref[...], sg_ref[...], su_ref[...],
                                   sd_ref[...])


def _gmm(be, xs, flat_packed, egw, euw, edw, sgw, suw, sdw):
    def wmap(b, be_ref):
        return (jnp.clip(be_ref[b], 0, E - 1), 0, 0)

    grid_spec = pltpu.PrefetchScalarGridSpec(
        num_scalar_prefetch=1,
        grid=(NBLKT,),
        in_specs=[
            pl.BlockSpec((BM, HW),
                         lambda b, be_ref: (jnp.minimum(b, NBLK - 1), 0)),
            pl.BlockSpec((BM, HW),
                         lambda b, be_ref: (jnp.maximum(b - NBLK, 0), 0)),
            pl.BlockSpec((1, FF, H), wmap),
            pl.BlockSpec((1, FF, H), wmap),
            pl.BlockSpec((1, H, FF), wmap),
            pl.BlockSpec((FF, H), lambda b, be_ref: (0, 0)),
            pl.BlockSpec((FF, H), lambda b, be_ref: (0, 0)),
            pl.BlockSpec((H, FF), lambda b, be_ref: (0, 0)),
        ],
        out_specs=[
            pl.BlockSpec((BM, HW),
                         lambda b, be_ref: (jnp.minimum(b, NBLK - 1), 0)),
            pl.BlockSpec((BM, HW),
                         lambda b, be_ref: (jnp.maximum(b - NBLK, 0), 0)),
        ],
    )
    return pl.pallas_call(
        _gmm_body, grid_spec=grid_spec,
        out_shape=[jax.ShapeDtypeStruct((MEXP, HW), I32),
                   jax.ShapeDtypeStruct((T, HW), I32)],
    )(be, xs, flat_packed, egw, euw, edw, sgw, suw, sdw)


_CT = 8
_C5 = (T // NW) // _CT


@functools.cache
def _build_combine():
    @functools.partial(
        pl.kernel, mesh=_sc_mesh(),
        out_type=jax.ShapeDtypeStruct((T, H), F32),
        scratch_types=[pltpu.VMEM((_C5, _CT), I32),
                       pltpu.VMEM((_C5, _CT), I32),
                       pltpu.VMEM((_C5 * _CT // 16, 16), F32),
                       pltpu.VMEM((_C5 * _CT // 16, 16), F32),
                       pltpu.VMEM((2, _CT, HW), I32),
                       pltpu.VMEM((2, _CT, HW), I32),
                       pltpu.VMEM((2, _CT, HW), I32),
                       pltpu.VMEM((2, _CT, H), F32),
                       pltpu.SemaphoreType.DMA,
                       pltpu.SemaphoreType.DMA,
                       pltpu.SemaphoreType.DMA],
    )
    def combine_kernel(ys_hbm, ysh_hbm, pe_hbm, po_hbm, we_hbm, wo_hbm, y_hbm,
                       idxe_v, idxo_v, we_v, wo_v, r0_v, r1_v, rsh_v, out_v,
                       sem0, sem1, sem_out):
        wid = lax.axis_index("s") * 2 + lax.axis_index("c")
        pltpu.sync_copy(pe_hbm.at[wid], idxe_v)
        pltpu.sync_copy(po_hbm.at[wid], idxo_v)
        pltpu.sync_copy(we_hbm.at[wid], we_v)
        pltpu.sync_copy(wo_hbm.at[wid], wo_v)
        sems = (sem0, sem1)

        def issue(c, slot):
            base = wid * (T // NW) + c * _CT
            return (
                pltpu.async_copy(ys_hbm.at[idxe_v.at[c]], r0_v.at[slot],
                                 sems[slot]),
                pltpu.async_copy(ys_hbm.at[idxo_v.at[c]], r1_v.at[slot],
                                 sems[slot]),
                pltpu.async_copy(ysh_hbm.at[pl.ds(base, _CT)], rsh_v.at[slot],
                                 sems[slot]),
            )

        handles = [issue(0, 0), None]
        out_handles = [None, None]
        for c in range(_C5):
            slot = c % 2
            if c + 1 < _C5:
                handles[1 - slot] = issue(c + 1, 1 - slot)
            for h in handles[slot]:
                h.wait()
            if out_handles[slot] is not None:
                out_handles[slot].wait()
            base = wid * (T // NW) + c * _CT
            w0r = we_v[c // 2, :]
            w1r = wo_v[c // 2, :]
            off = (c % 2) * _CT

            def tok(i, _):
                sel = jnp.zeros((16,), I32) + (off + i)
                b0 = jnp.take(w0r, sel)
                b1 = jnp.take(w1r, sel)

                def col(j, _):
                    sl = pl.ds(j * 16, 16)
                    r0lo, r0hi = _unpack_row(r0_v[slot, i, sl])
                    r1lo, r1hi = _unpack_row(r1_v[slot, i, sl])
                    shlo, shhi = _unpack_row(rsh_v[slot, i, sl])
                    out_v[slot, i, sl] = b0 * r0lo + b1 * r1lo + shlo
                    hi_sl = pl.ds(HW + j * 16, 16)
                    out_v[slot, i, hi_sl] = b0 * r0hi + b1 * r1hi + shhi
                    return 0

                lax.fori_loop(0, HW // 16, col, 0, unroll=8)
                return 0

            lax.fori_loop(0, _CT, tok, 0)
            out_handles[slot] = pltpu.async_copy(
                out_v.at[slot], y_hbm.at[pl.ds(base, _CT)], sem_out)
        for oh_ in out_handles:
            if oh_ is not None:
                oh_.wait()

    return combine_kernel


def _combine(ys, ysh, pe5, po5, w05, w15):
    return _build_combine()(ys, ysh, pe5, po5, w05, w15)


def kernel(hidden_states, gate_weight, expert_gate_w, expert_up_w,
           expert_down_w, shared_gate_w, shared_up_w, shared_down_w):
    flat = hidden_states.reshape(T, H)
    logits_t, tidx_t, tw_t, flat_bf = _route(flat, gate_weight)
    logits = logits_t.T
    tidx = tidx_t[:K].T
    tw = tw_t[:K].T
    pos_col, be_col = _dispatch(tidx.reshape(P, 1))
    pos = pos_col.reshape(T, K)
    be = be_col.reshape(NBLKT)
    pe3 = pos[:, 0].reshape(NW, _C3, 32)
    po3 = pos[:, 1].reshape(NW, _C3, 32)
    xs = _scatter(flat_bf, pe3, po3)
    ys, ysh = _gmm(be, xs, flat_bf, expert_gate_w, expert_up_w, expert_down_w,
                   shared_gate_w, shared_up_w, shared_down_w)
    pe5 = pos[:, 0].reshape(NW, _C5, _CT)
    po5 = pos[:, 1].reshape(NW, _C5, _CT)
    w05 = tw[:, 0].reshape(NW, _C5 * _CT // 16, 16)
    w15 = tw[:, 1].reshape(NW, _C5 * _CT // 16, 16)
    y = _combine(ys, ysh, pe5, po5, w05, w15)
    return y.reshape(B, S, H), (logits.reshape(B, S, E), tidx.reshape(B, S, K))


# speedup vs baseline: 4.9522x; 1.0304x over previous
import functools

import jax
import jax.numpy as jnp
from jax import lax
from jax.experimental import pallas as pl
from jax.experimental.pallas import tpu as pltpu
from jax.experimental.pallas import tpu_sc as plsc

F32 = jnp.float32
I32 = jnp.int32
NEG = -1e30

B, S, H = 2, 2048, 1024
E, K = 16, 2
FF = 512
NG = 4
GS = E // NG
T = B * S
P = T * K
BM = 256
BMLOG = 8
NBLK = (P + E * (BM - 1) + BM - 1) // BM
MEXP = NBLK * BM
NBLKT = NBLK + T // BM
RB = 512
PB = 1024
NPB = P // PB
NW = 32
HW = H // 2


def _pack_row(xlo, xhi):
    ulo = lax.bitcast_convert_type(xlo, jnp.uint32)
    uhi = lax.bitcast_convert_type(xhi, jnp.uint32)
    rlo = ulo + jnp.uint32(0x7FFF) + ((ulo >> 16) & jnp.uint32(1))
    rhi = uhi + jnp.uint32(0x7FFF) + ((uhi >> 16) & jnp.uint32(1))
    w = (rlo >> 16) | (rhi & jnp.uint32(0xFFFF0000))
    return lax.bitcast_convert_type(w, I32)


def _unpack_row(w):
    lo = lax.bitcast_convert_type(w << 16, F32)
    hi = lax.bitcast_convert_type(w & jnp.int32(-65536), F32)
    return lo, hi


def _routing_body(x_ref, gw_ref, logits_ref, tidx_ref, tw_ref, xbf_ref):
    x = x_ref[...]
    gw = gw_ref[...]
    logits = lax.dot_general(x, gw, (((1,), (1,)), ((), ())),
                             preferred_element_type=F32)
    lt = jnp.transpose(logits)
    logits_ref[...] = lt
    xbf_ref[...] = _pack_row(x[:, :HW], x[:, HW:])

    scores = 1.0 / (1.0 + jnp.exp(-lt))
    iota = lax.broadcasted_iota(I32, (E, RB), 0)
    grp = iota // GS

    m1 = jnp.full((E, RB), NEG)
    for g in range(NG):
        mg = jnp.max(scores[g * GS:(g + 1) * GS], axis=0, keepdims=True)
        m1 = jnp.where(grp == g, mg, m1)
    cand = jnp.where(scores == m1, iota, E)
    fst = jnp.zeros((E, RB), I32)
    for g in range(NG):
        fg = jnp.min(cand[g * GS:(g + 1) * GS], axis=0, keepdims=True)
        fst = jnp.where(grp == g, fg, fst)
    removed = iota == fst
    scores2 = jnp.where(removed, NEG, scores)
    m2 = jnp.full((E, RB), NEG)
    for g in range(NG):
        mg = jnp.max(scores2[g * GS:(g + 1) * GS], axis=0, keepdims=True)
        m2 = jnp.where(grp == g, mg, m2)
    gsc = m1 + m2

    rep = iota % GS == 0
    gsc_rep = jnp.where(rep, gsc, NEG)
    g1v = jnp.max(gsc_rep, axis=0, keepdims=True)
    g1row = jnp.min(jnp.where(gsc_rep == g1v, iota, E), axis=0, keepdims=True)
    gsc_rep2 = jnp.where(iota == g1row, NEG, gsc_rep)
    g2v = jnp.max(gsc_rep2, axis=0, keepdims=True)
    g2row = jnp.min(jnp.where(gsc_rep2 == g2v, iota, E), axis=0, keepdims=True)
    gmask = (grp == g1row // GS) | (grp == g2row // GS)

    masked = jnp.where(gmask, scores, NEG)
    v1 = jnp.max(masked, axis=0, keepdims=True)
    e1 = jnp.min(jnp.where(masked == v1, iota, E), axis=0, keepdims=True)
    masked2 = jnp.where(iota == e1, NEG, masked)
    v2 = jnp.max(masked2, axis=0, keepdims=True)
    e2 = jnp.min(jnp.where(masked2 == v2, iota, E), axis=0, keepdims=True)
    denom = v1 + v2 + 1e-20

    rowi = lax.broadcasted_iota(I32, (8, RB), 0)
    tidx_ref[...] = jnp.where(rowi == 0, e1, jnp.where(rowi == 1, e2, 0))
    tw_ref[...] = jnp.where(rowi == 0, v1 / denom,
                            jnp.where(rowi == 1, v2 / denom, 0.0))


def _route(flat, gW):
    return pl.pallas_call(
        _routing_body,
        grid=(T // RB,),
        in_specs=[pl.BlockSpec((RB, H), lambda b: (b, 0)),
                  pl.BlockSpec((E, H), lambda b: (0, 0))],
        out_specs=[pl.BlockSpec((E, RB), lambda b: (0, b)),
                   pl.BlockSpec((8, RB), lambda b: (0, b)),
                   pl.BlockSpec((8, RB), lambda b: (0, b)),
                   pl.BlockSpec((RB, HW), lambda b: (b, 0))],
        out_shape=[jax.ShapeDtypeStruct((E, T), F32),
                   jax.ShapeDtypeStruct((8, T), I32),
                   jax.ShapeDtypeStruct((8, T), F32),
                   jax.ShapeDtypeStruct((T, HW), I32)],
    )(flat, gW)


def _dispatch_body(e_ref, pos_ref, be_ref, acc, bs, po, tot, tril_s):
    s = pl.program_id(0)
    b = pl.program_id(1)
    e = e_ref[...]
    iota = lax.broadcasted_iota(I32, (PB, E), 1)
    oh = jnp.where(e == iota, 1.0, 0.0).astype(F32)

    @pl.when((s == 0) & (b == 0))
    def _():
        acc[...] = jnp.zeros_like(acc)
        r = lax.broadcasted_iota(I32, (PB, PB), 0)
        c = lax.broadcasted_iota(I32, (PB, PB), 1)
        tril_s[...] = jnp.where(r > c, 1.0, 0.0).astype(jnp.bfloat16)

    @pl.when(s == 0)
    def _():
        bs[pl.ds(b, 1), :] = acc[0:1, :]
        acc[0:1, :] = acc[0:1, :] + jnp.sum(oh, axis=0, keepdims=True)

    @pl.when((s == 1) & (b == 0))
    def _():
        ci = acc[0:1, :].astype(I32)
        pc = ((ci + (BM - 1)) >> BMLOG) << BMLOG
        pcf = pc.astype(F32)
        r = lax.broadcasted_iota(I32, (E, E), 0)
        c = lax.broadcasted_iota(I32, (E, E), 1)
        m = jnp.where(r < c, 1.0, 0.0)
        po[0:1, :] = lax.dot_general(pcf, m, (((1,), (0,)), ((), ())),
                                     preferred_element_type=F32,
                                     precision=lax.Precision.HIGHEST)
        tot[0:1, :] = jnp.broadcast_to(jnp.sum(pcf, axis=1, keepdims=True),
                                       (1, E))

    @pl.when(s == 1)
    def _():
        ranks = lax.dot_general(tril_s[...], oh.astype(jnp.bfloat16),
                                (((1,), (0,)), ((), ())),
                                preferred_element_type=F32)
        rank = jnp.sum(ranks * oh, axis=1, keepdims=True)
        base = jnp.sum(oh * (po[0:1, :] + bs[pl.ds(b, 1), :]),
                       axis=1, keepdims=True)
        pos_ref[...] = (rank + base).astype(I32)
        bi = lax.broadcasted_iota(I32, (NBLKT, E), 0).astype(F32) * BM
        nle = jnp.sum(jnp.where(bi >= po[0:1, :], 1.0, 0.0),
                      axis=1, keepdims=True)
        bstart = lax.broadcasted_iota(I32, (NBLKT, 1), 0).astype(F32) * BM
        rowi = lax.broadcasted_iota(I32, (NBLKT, 1), 0)
        be_ref[...] = jnp.where(
            rowi >= NBLK, E,
            jnp.where(bstart < tot[0:1, 0:1], nle - 1.0, -1.0).astype(I32))


def _dispatch(e_col):
    return pl.pallas_call(
        _dispatch_body,
        grid=(2, NPB),
        in_specs=[pl.BlockSpec((PB, 1), lambda s, b: (b, 0))],
        out_specs=[pl.BlockSpec((PB, 1), lambda s, b: (b, 0)),
                   pl.BlockSpec((NBLKT, 1), lambda s, b: (0, 0))],
        out_shape=[jax.ShapeDtypeStruct((P, 1), I32),
                   jax.ShapeDtypeStruct((NBLKT, 1), I32)],
        scratch_shapes=[pltpu.VMEM((8, E), F32),
                        pltpu.VMEM((NPB, E), F32),
                        pltpu.VMEM((8, E), F32),
                        pltpu.VMEM((8, E), F32),
                        pltpu.VMEM((PB, PB), jnp.bfloat16)],
    )(e_col)


_C3 = 4


@functools.cache
def _sc_mesh():
    return plsc.VectorSubcoreMesh(core_axis_name="c", subcore_axis_name="s",
                                  num_cores=2, num_subcores=16)


@functools.cache
def _build_scatter():
    @functools.partial(
        pl.kernel, mesh=_sc_mesh(),
        out_type=jax.ShapeDtypeStruct((MEXP, HW), I32),
        scratch_types=[pltpu.VMEM((_C3, 32), I32),
                       pltpu.VMEM((_C3, 32), I32),
                       pltpu.VMEM((32, HW), I32),
                       pltpu.SemaphoreType.DMA],
    )
    def scatter_kernel(flat_hbm, pe_hbm, po_hbm, xs_hbm,
                       idxe_v, idxo_v, rows_v, sem):
        wid = lax.axis_index("s") * 2 + lax.axis_index("c")
        pltpu.sync_copy(pe_hbm.at[wid], idxe_v)
        pltpu.sync_copy(po_hbm.at[wid], idxo_v)
        for c in range(_C3):
            base = wid * (T // NW) + c * 32
            pltpu.sync_copy(flat_hbm.at[pl.ds(base, 32)], rows_v)
            pltpu.async_copy(rows_v, xs_hbm.at[idxe_v.at[c]], sem).wait()
            pltpu.async_copy(rows_v, xs_hbm.at[idxo_v.at[c]], sem).wait()

    return scatter_kernel


def _scatter(flat, pe3, po3):
    return _build_scatter()(flat, pe3, po3)


def _mlp_packed(xw, gw, uw, dw):
    xlo_f, xhi_f = _unpack_row(xw)
    xlo = xlo_f.astype(jnp.bfloat16)
    xhi = xhi_f.astype(jnp.bfloat16)
    dn = (((1,), (1,)), ((), ()))
    gwb = gw.astype(jnp.bfloat16)
    uwb = uw.astype(jnp.bfloat16)
    g = (lax.dot_general(xlo, gwb[:, :HW], dn, preferred_element_type=F32)
         + lax.dot_general(xhi, gwb[:, HW:], dn, preferred_element_type=F32))
    u = (lax.dot_general(xlo, uwb[:, :HW], dn, preferred_element_type=F32)
         + lax.dot_general(xhi, uwb[:, HW:], dn, preferred_element_type=F32))
    h = (g / (1.0 + jnp.exp(-g)) * u).astype(jnp.bfloat16)
    y = lax.dot_general(h, dw.astype(jnp.bfloat16), dn,
                        preferred_element_type=F32)
    return _pack_row(y[:, :HW], y[:, HW:])


def _gmm_body(be_ref, x_ref, fl_ref, gw_ref, uw_ref, dw_ref,
              sg_ref, su_ref, sd_ref, o_ref, osh_ref):
    b = pl.program_id(0)
    e = be_ref[b]

    @pl.when((e >= 0) & (e < E))
    def _():
        o_ref[...] = _mlp_packed(x_ref[...], gw_ref[0], uw_ref[0], dw_ref[0])

    @pl.when(e == E)
    def _():
        osh_ref[...] = _mlp_packed(fl_ref[...], sg_ref[...], su_ref[...],
                                   sd_ref[...])


def _gmm(be, xs, flat_packed, egw, euw, edw, sgw, suw, sdw):
    def wmap(b, be_ref):
        return (jnp.clip(be_ref[b], 0, E - 1), 0, 0)

    grid_spec = pltpu.PrefetchScalarGridSpec(
        num_scalar_prefetch=1,
        grid=(NBLKT,),
        in_specs=[
            pl.BlockSpec((BM, HW),
                         lambda b, be_ref: (jnp.minimum(b, NBLK - 1), 0)),
            pl.BlockSpec((BM, HW),
                         lambda b, be_ref: (jnp.maximum(b - NBLK, 0), 0)),
            pl.BlockSpec((1, FF, H), wmap),
            pl.BlockSpec((1, FF, H), wmap),
            pl.BlockSpec((1, H, FF), wmap),
            pl.BlockSpec((FF, H), lambda b, be_ref: (0, 0)),
            pl.BlockSpec((FF, H), lambda b, be_ref: (0, 0)),
            pl.BlockSpec((H, FF), lambda b, be_ref: (0, 0)),
        ],
        out_specs=[
            pl.BlockSpec((BM, HW),
                         lambda b, be_ref: (jnp.minimum(b, NBLK - 1), 0)),
            pl.BlockSpec((BM, HW),
                         lambda b, be_ref: (jnp.maximum(b - NBLK, 0), 0)),
        ],
    )
    return pl.pallas_call(
        _gmm_body, grid_spec=grid_spec,
        out_shape=[jax.ShapeDtypeStruct((MEXP, HW), I32),
                   jax.ShapeDtypeStruct((T, HW), I32)],
    )(be, xs, flat_packed, egw, euw, edw, sgw, suw, sdw)


_CT = 16
_C5 = (T // NW) // _CT


@functools.cache
def _build_combine():
    @functools.partial(
        pl.kernel, mesh=_sc_mesh(),
        out_type=jax.ShapeDtypeStruct((T, H), F32),
        scratch_types=[pltpu.VMEM((_C5, _CT), I32),
                       pltpu.VMEM((_C5, _CT), I32),
                       pltpu.VMEM((_C5 * _CT // 16, 16), F32),
                       pltpu.VMEM((_C5 * _CT // 16, 16), F32),
                       pltpu.VMEM((2, _CT, HW), I32),
                       pltpu.VMEM((2, _CT, HW), I32),
                       pltpu.VMEM((2, _CT, HW), I32),
                       pltpu.VMEM((2, _CT, H), F32),
                       pltpu.SemaphoreType.DMA,
                       pltpu.SemaphoreType.DMA,
                       pltpu.SemaphoreType.DMA],
    )
    def combine_kernel(ys_hbm, ysh_hbm, pe_hbm, po_hbm, we_hbm, wo_hbm, y_hbm,
                       idxe_v, idxo_v, we_v, wo_v, r0_v, r1_v, rsh_v, out_v,
                       sem0, sem1, sem_out):
        wid = lax.axis_index("s") * 2 + lax.axis_index("c")
        pltpu.sync_copy(pe_hbm.at[wid], idxe_v)
        pltpu.sync_copy(po_hbm.at[wid], idxo_v)
        pltpu.sync_copy(we_hbm.at[wid], we_v)
        pltpu.sync_copy(wo_hbm.at[wid], wo_v)
        sems = (sem0, sem1)

        def issue(c, slot):
            base = wid * (T // NW) + c * _CT
            return (
                pltpu.async_copy(ys_hbm.at[idxe_v.at[c]], r0_v.at[slot],
                                 sems[slot]),
                pltpu.async_copy(ys_hbm.at[idxo_v.at[c]], r1_v.at[slot],
                                 sems[slot]),
                pltpu.async_copy(ysh_hbm.at[pl.ds(base, _CT)], rsh_v.at[slot],
                                 sems[slot]),
            )

        handles = [issue(0, 0), None]
        out_handles = [None, None]
        for c in range(_C5):
            slot = c % 2
            if c + 1 < _C5:
                handles[1 - slot] = issue(c + 1, 1 - slot)
            for h in handles[slot]:
                h.wait()
            if out_handles[slot] is not None:
                out_handles[slot].wait()
            base = wid * (T // NW) + c * _CT
            w0r = we_v[c, :]
            w1r = wo_v[c, :]

            def tok(i, _):
                sel = jnp.zeros((16,), I32) + i
                b0 = jnp.take(w0r, sel)
                b1 = jnp.take(w1r, sel)

                def col(j, _):
                    sl = pl.ds(j * 16, 16)
                    r0lo, r0hi = _unpack_row(r0_v[slot, i, sl])
                    r1lo, r1hi = _unpack_row(r1_v[slot, i, sl])
                    shlo, shhi = _unpack_row(rsh_v[slot, i, sl])
                    out_v[slot, i, sl] = b0 * r0lo + b1 * r1lo + shlo
                    hi_sl = pl.ds(HW + j * 16, 16)
                    out_v[slot, i, hi_sl] = b0 * r0hi + b1 * r1hi + shhi
                    return 0

                lax.fori_loop(0, HW // 16, col, 0, unroll=8)
                return 0

            lax.fori_loop(0, _CT, tok, 0)
            out_handles[slot] = pltpu.async_copy(
                out_v.at[slot], y_hbm.at[pl.ds(base, _CT)], sem_out)
        for oh_ in out_handles:
            if oh_ is not None:
                oh_.wait()

    return combine_kernel


def _combine(ys, ysh, pe5, po5, w05, w15):
    return _build_combine()(ys, ysh, pe5, po5, w05, w15)


def kernel(hidden_states, gate_weight, expert_gate_w, expert_up_w,
           expert_down_w, shared_gate_w, shared_up_w, shared_down_w):
    flat = hidden_states.reshape(T, H)
    logits_t, tidx_t, tw_t, flat_bf = _route(flat, gate_weight)
    logits = logits_t.T
    tidx = tidx_t[:K].T
    tw = tw_t[:K].T
    pos_col, be_col = _dispatch(tidx.reshape(P, 1))
    pos = pos_col.reshape(T, K)
    be = be_col.reshape(NBLKT)
    pe3 = pos[:, 0].reshape(NW, _C3, 32)
    po3 = pos[:, 1].reshape(NW, _C3, 32)
    xs = _scatter(flat_bf, pe3, po3)
    ys, ysh = _gmm(be, xs, flat_bf, expert_gate_w, expert_up_w, expert_down_w,
                   shared_gate_w, shared_up_w, shared_down_w)
    pe5 = pos[:, 0].reshape(NW, _C5, _CT)
    po5 = pos[:, 1].reshape(NW, _C5, _CT)
    w05 = tw[:, 0].reshape(NW, _C5 * _CT // 16, 16)
    w15 = tw[:, 1].reshape(NW, _C5 * _CT // 16, 16)
    y = _combine(ys, ysh, pe5, po5, w05, w15)
    return y.reshape(B, S, H), (logits.reshape(B, S, E), tidx.reshape(B, S, K))
